# baseline (speedup 1.0000x reference)
import functools

import jax
import jax.numpy as jnp
from jax import lax
from jax.experimental import pallas as pl
from jax.experimental.pallas import tpu as pltpu
from jax.experimental.pallas import tpu_sc as plsc

N_V = 50000
N_E = 800000
N_G = 512
D = 32

_E_W = 100
_E_K = 8
_E_ROWS = _E_W * _E_K
_E_CHUNKS = N_E // _E_ROWS
_N_W = 25
_N_K = 8
_N_ROWS = _N_W * _N_K
_N_CHUNKS = N_V // _N_ROWS

_NW = 32
_E_TRIPS = (_E_CHUNKS + _NW - 1) // _NW
_N_TRIPS = (_N_CHUNKS + _NW - 1) // _NW

_MESH = plsc.VectorSubcoreMesh(core_axis_name="c", subcore_axis_name="s")
_NSUB = 16
_VSUB = 10
_VROWS = N_V // _VSUB
_GROWS = N_G // _NSUB


def _worker_id():
    return lax.axis_index("s") * 2 + lax.axis_index("c")


@functools.partial(
    pl.kernel,
    out_type=(
        jax.ShapeDtypeStruct((N_E, D), jnp.float32),
        jax.ShapeDtypeStruct((N_E, D), jnp.float32),
        jax.ShapeDtypeStruct((N_E, D), jnp.float32),
        jax.ShapeDtypeStruct((N_V, D), jnp.float32),
    ),
    mesh=_MESH,
    compiler_params=pltpu.CompilerParams(use_tc_tiling_on_sc=False),
    scratch_types=[
        pltpu.VMEM_SHARED((N_G, D), jnp.float32),
        pltpu.VMEM((_E_K, _E_W), jnp.int32),
        pltpu.VMEM((_E_K, _E_W), jnp.int32),
        pltpu.VMEM((_E_K, _E_W), jnp.int32),
        pltpu.VMEM((4, _E_W, D), jnp.float32),
        pltpu.VMEM((4, _E_W, D), jnp.float32),
        pltpu.VMEM((4, _E_W, D), jnp.float32),
        pltpu.VMEM((_N_K, _N_W), jnp.int32),
        pltpu.VMEM((4, _N_W, D), jnp.float32),
        pltpu.SemaphoreType.DMA,
        pltpu.SemaphoreType.DMA,
    ],
)
def _sc_gather(xv_hbm, xu_hbm, src_hbm, dst_hbm, bb_hbm, nb_hbm,
               os_hbm, od_hbm, ob_hbm, on_hbm,
               xu_spm, si, di, bi, gs, gd, gb, ni, gn, gsem, wsem):
    sid = lax.axis_index("s")
    wid = _worker_id()

    pltpu.sync_copy(xu_hbm.at[pl.ds(sid * _GROWS, _GROWS)],
                    xu_spm.at[pl.ds(sid * _GROWS, _GROWS)])
    plsc.subcore_barrier()

    @pl.loop(0, _E_TRIPS)
    def _(t):
        c = wid + t * _NW

        @pl.when(c < _E_CHUNKS)
        def _():
            base = c * _E_ROWS
            pltpu.sync_copy(src_hbm.at[pl.ds(c * _E_K, _E_K)], si)
            pltpu.sync_copy(dst_hbm.at[pl.ds(c * _E_K, _E_K)], di)
            pltpu.sync_copy(bb_hbm.at[pl.ds(c * _E_K, _E_K)], bi)

            w = {}
            for j in range(_E_K):
                b = j & 3
                ww = pl.ds(base + j * _E_W, _E_W)
                if j >= 2:
                    for h in w.pop(j - 2):
                        h.wait()
                pltpu.async_copy(xv_hbm.at[si.at[j]], gs.at[b], gsem).wait()
                pltpu.async_copy(xv_hbm.at[di.at[j]], gd.at[b], gsem).wait()
                pltpu.async_copy(xu_spm.at[bi.at[j]], gb.at[b], gsem).wait()
                w[j] = (
                    pltpu.async_copy(gs.at[b], os_hbm.at[ww], wsem),
                    pltpu.async_copy(gd.at[b], od_hbm.at[ww], wsem),
                    pltpu.async_copy(gb.at[b], ob_hbm.at[ww], wsem),
                )
            for j in (_E_K - 2, _E_K - 1):
                for h in w.pop(j):
                    h.wait()

    @pl.loop(0, _N_TRIPS)
    def _(t):
        c = wid + t * _NW

        @pl.when(c < _N_CHUNKS)
        def _():
            base = c * _N_ROWS
            pltpu.sync_copy(nb_hbm.at[pl.ds(c * _N_K, _N_K)], ni)

            w = {}
            for j in range(_N_K):
                if j >= 2:
                    w.pop(j - 2).wait()
                pltpu.async_copy(xu_spm.at[ni.at[j]], gn.at[j & 3],
                                 gsem).wait()
                w[j] = pltpu.async_copy(
                    gn.at[j & 3], on_hbm.at[pl.ds(base + j * _N_W, _N_W)],
                    wsem)
            for j in (_N_K - 2, _N_K - 1):
                w.pop(j).wait()


@functools.partial(
    pl.kernel,
    out_type=(
        jax.ShapeDtypeStruct((2, N_V, D), jnp.float32),
        jax.ShapeDtypeStruct((2, N_G, D), jnp.float32),
    ),
    mesh=_MESH,
    compiler_params=pltpu.CompilerParams(use_tc_tiling_on_sc=False),
    scratch_types=[
        pltpu.VMEM_SHARED((N_V, D), jnp.float32),
        pltpu.VMEM_SHARED((N_G, D), jnp.float32),
        pltpu.VMEM((_E_K, _E_W), jnp.int32),
        pltpu.VMEM((_E_K, _E_W), jnp.int32),
        pltpu.VMEM((4, _E_W, D), jnp.float32),
        pltpu.SemaphoreType.DMA,
        pltpu.SemaphoreType.DMA,
    ],
)
def _sc_scatter_edges(e_hbm, dst_hbm, bb_hbm, z_hbm, ov_hbm, ou_hbm,
                      accv, accu, di, bi, ge, lsem, ssem):
    cid = lax.axis_index("c")
    sid = lax.axis_index("s")
    wid = _worker_id()

    @pl.when(sid < _VSUB)
    def _():
        pltpu.sync_copy(z_hbm.at[pl.ds(sid * _VROWS, _VROWS)],
                        accv.at[pl.ds(sid * _VROWS, _VROWS)])

    pltpu.sync_copy(z_hbm.at[pl.ds(sid * _GROWS, _GROWS)],
                    accu.at[pl.ds(sid * _GROWS, _GROWS)])
    plsc.subcore_barrier()

    @pl.loop(0, _E_TRIPS)
    def _(t):
        c = wid + t * _NW

        @pl.when(c < _E_CHUNKS)
        def _():
            base = c * _E_ROWS
            pltpu.sync_copy(dst_hbm.at[pl.ds(c * _E_K, _E_K)], di)
            pltpu.sync_copy(bb_hbm.at[pl.ds(c * _E_K, _E_K)], bi)

            def load(j):
                return pltpu.async_copy(
                    e_hbm.at[pl.ds(base + j * _E_W, _E_W)], ge.at[j & 3],
                    lsem)

            ld = {0: load(0), 1: load(1)}
            for j in range(_E_K):
                b = j & 3
                ld.pop(j).wait()
                if j + 2 < _E_K:
                    ld[j + 2] = load(j + 2)
                pltpu.async_copy(ge.at[b], accv.at[di.at[j]], ssem,
                                 add=True).wait()
                pltpu.async_copy(ge.at[b], accu.at[bi.at[j]], ssem,
                                 add=True).wait()

    plsc.subcore_barrier()

    @pl.when(sid < _VSUB)
    def _():
        pltpu.sync_copy(accv.at[pl.ds(sid * _VROWS, _VROWS)],
                        ov_hbm.at[cid, pl.ds(sid * _VROWS, _VROWS)])

    pltpu.sync_copy(accu.at[pl.ds(sid * _GROWS, _GROWS)],
                    ou_hbm.at[cid, pl.ds(sid * _GROWS, _GROWS)])


@functools.partial(
    pl.kernel,
    out_type=jax.ShapeDtypeStruct((2, N_G, D), jnp.float32),
    mesh=_MESH,
    compiler_params=pltpu.CompilerParams(use_tc_tiling_on_sc=False),
    scratch_types=[
        pltpu.VMEM_SHARED((N_G, D), jnp.float32),
        pltpu.VMEM((_N_K, _N_W), jnp.int32),
        pltpu.VMEM((4, _N_W, D), jnp.float32),
        pltpu.SemaphoreType.DMA,
        pltpu.SemaphoreType.DMA,
    ],
)
def _sc_scatter_nodes(v_hbm, nb_hbm, z_hbm, ou_hbm, accu, ni, gv, lsem, ssem):
    cid = lax.axis_index("c")
    sid = lax.axis_index("s")
    wid = _worker_id()
    pltpu.sync_copy(z_hbm.at[pl.ds(sid * _GROWS, _GROWS)],
                    accu.at[pl.ds(sid * _GROWS, _GROWS)])
    plsc.subcore_barrier()

    @pl.loop(0, _N_TRIPS)
    def _(t):
        c = wid + t * _NW

        @pl.when(c < _N_CHUNKS)
        def _():
            base = c * _N_ROWS
            pltpu.sync_copy(nb_hbm.at[pl.ds(c * _N_K, _N_K)], ni)

            def load(j):
                return pltpu.async_copy(
                    v_hbm.at[pl.ds(base + j * _N_W, _N_W)], gv.at[j & 3],
                    lsem)

            ld = {0: load(0), 1: load(1)}
            for j in range(_N_K):
                ld.pop(j).wait()
                if j + 2 < _N_K:
                    ld[j + 2] = load(j + 2)
                pltpu.async_copy(gv.at[j & 3], accu.at[ni.at[j]], ssem,
                                 add=True).wait()

    plsc.subcore_barrier()
    pltpu.sync_copy(accu.at[pl.ds(sid * _GROWS, _GROWS)],
                    ou_hbm.at[cid, pl.ds(sid * _GROWS, _GROWS)])


@functools.partial(
    pl.kernel,
    out_type=(
        jax.ShapeDtypeStruct((2, N_V), jnp.float32),
        jax.ShapeDtypeStruct((2, N_G), jnp.float32),
        jax.ShapeDtypeStruct((2, N_G), jnp.float32),
    ),
    mesh=_MESH,
    compiler_params=pltpu.CompilerParams(use_tc_tiling_on_sc=False),
    scratch_types=[
        pltpu.VMEM_SHARED((N_V,), jnp.float32),
        pltpu.VMEM_SHARED((N_G,), jnp.float32),
        pltpu.VMEM_SHARED((N_G,), jnp.float32),
        pltpu.VMEM((_E_K, _E_W), jnp.int32),
        pltpu.VMEM((_E_K, _E_W), jnp.int32),
        pltpu.VMEM((_N_K, _N_W), jnp.int32),
        pltpu.VMEM((128,), jnp.float32),
        pltpu.SemaphoreType.DMA,
    ],
)
def _sc_counts(dst_hbm, bb_hbm, nb_hbm, z_hbm, od_hbm, oe_hbm, ov_hbm,
               accd, acce, accv, di, bi, ni, ones, ssem):
    cid = lax.axis_index("c")
    sid = lax.axis_index("s")
    wid = _worker_id()
    for j in range(8):
        ones[pl.ds(j * 16, 16)] = jnp.ones((16,), jnp.float32)

    @pl.when(sid < _VSUB)
    def _():
        pltpu.sync_copy(z_hbm.at[pl.ds(sid * _VROWS, _VROWS)],
                        accd.at[pl.ds(sid * _VROWS, _VROWS)])

    pltpu.sync_copy(z_hbm.at[pl.ds(sid * _GROWS, _GROWS)],
                    acce.at[pl.ds(sid * _GROWS, _GROWS)])
    pltpu.sync_copy(z_hbm.at[pl.ds(sid * _GROWS, _GROWS)],
                    accv.at[pl.ds(sid * _GROWS, _GROWS)])
    plsc.subcore_barrier()

    @pl.loop(0, _E_TRIPS)
    def _(t):
        c = wid + t * _NW

        @pl.when(c < _E_CHUNKS)
        def _():
            pltpu.sync_copy(dst_hbm.at[pl.ds(c * _E_K, _E_K)], di)
            pltpu.sync_copy(bb_hbm.at[pl.ds(c * _E_K, _E_K)], bi)
            for j in range(_E_K):
                pltpu.async_copy(ones.at[pl.ds(0, _E_W)], accd.at[di.at[j]],
                                 ssem, add=True).wait()
                pltpu.async_copy(ones.at[pl.ds(0, _E_W)], acce.at[bi.at[j]],
                                 ssem, add=True).wait()

    @pl.loop(0, _N_TRIPS)
    def _(t):
        c = wid + t * _NW

        @pl.when(c < _N_CHUNKS)
        def _():
            pltpu.sync_copy(nb_hbm.at[pl.ds(c * _N_K, _N_K)], ni)
            for j in range(_N_K):
                pltpu.async_copy(ones.at[pl.ds(0, _N_W)], accv.at[ni.at[j]],
                                 ssem, add=True).wait()

    plsc.subcore_barrier()

    @pl.when(sid < _VSUB)
    def _():
        pltpu.sync_copy(accd.at[pl.ds(sid * _VROWS, _VROWS)],
                        od_hbm.at[cid, pl.ds(sid * _VROWS, _VROWS)])

    pltpu.sync_copy(acce.at[pl.ds(sid * _GROWS, _GROWS)],
                    oe_hbm.at[cid, pl.ds(sid * _GROWS, _GROWS)])
    pltpu.sync_copy(accv.at[pl.ds(sid * _GROWS, _GROWS)],
                    ov_hbm.at[cid, pl.ds(sid * _GROWS, _GROWS)])


_R = 2000


def _full(a):
    if a.ndim == 3:
        return pl.BlockSpec(a.shape, lambda i: (0, 0, 0))
    if a.ndim == 2:
        return pl.BlockSpec(a.shape, lambda i: (0, 0))
    return pl.BlockSpec(a.shape, lambda i: (0,))


def _sp(x):
    return jax.nn.softplus(x)


def _pre_v_kernel(x_ref, w1, b1, w2, b2, o_ref):
    h = _sp(x_ref[...] @ w1[...] + b1[...])
    o_ref[...] = _sp(h @ w2[...] + b2[...])


def _pre_v(x, p):
    w1, b1 = p[0]["W"], p[0]["b"][None]
    w2, b2 = p[1]["W"], p[1]["b"][None]
    din = x.shape[1]
    return pl.pallas_call(
        _pre_v_kernel,
        grid=(N_V // _R,),
        in_specs=[pl.BlockSpec((_R, din), lambda i: (i, 0))]
        + [_full(w) for w in (w1, b1, w2, b2)],
        out_specs=pl.BlockSpec((_R, D), lambda i: (i, 0)),
        out_shape=jax.ShapeDtypeStruct((N_V, D), jnp.float32),
    )(x, w1, b1, w2, b2)


def _pre_u_kernel(u_ref, w1, b1, w2, b2, o_ref):
    h = _sp(u_ref[...] @ w1[...] + b1[...])
    o_ref[...] = _sp(h @ w2[...] + b2[...])


def _pre_u(u, p):
    w1, b1 = p[0]["W"], p[0]["b"][None]
    w2, b2 = p[1]["W"], p[1]["b"][None]
    return pl.pallas_call(
        _pre_u_kernel,
        out_shape=jax.ShapeDtypeStruct((N_G, D), jnp.float32),
    )(u, w1, b1, w2, b2)


def _edge_body(ea, xvs, xvd, xub, w1, b1, w2, b2, wa, wb, wc, wd, bp, v2, c2,
               v3, c3):
    xe = _sp(ea[...] @ w1[...] + b1[...])
    xe = _sp(xe @ w2[...] + b2[...])
    h = _sp(xvs[...] @ wa[...] + xvd[...] @ wb[...] + xe @ wc[...]
            + xub[...] @ wd[...] + bp[...])
    h = _sp(h @ v2[...] + c2[...])
    return _sp(h @ v3[...] + c3[...])


def _edge_kernel_skip(ea, xvs, xvd, xub, w1, b1, w2, b2, wa, wb, wc, wd, bp,
                      v2, c2, v3, c3, o_pre, o_post):
    e_pre = _edge_body(ea, xvs, xvd, xub, w1, b1, w2, b2, wa, wb, wc, wd, bp,
                       v2, c2, v3, c3)
    o_pre[...] = e_pre
    o_post[...] = e_pre + ea[...]


def _edge_kernel_noskip(ea, xvs, xvd, xub, w1, b1, w2, b2, wa, wb, wc, wd, bp,
                        v2, c2, v3, c3, o_pre):
    o_pre[...] = _edge_body(ea, xvs, xvd, xub, w1, b1, w2, b2, wa, wb, wc, wd,
                            bp, v2, c2, v3, c3)


def _edge_mlp(ea, xvs, xvd, xub, p, skip_out):
    w1, b1 = p["pre_e"][0]["W"], p["pre_e"][0]["b"][None]
    w2, b2 = p["pre_e"][1]["W"], p["pre_e"][1]["b"][None]
    W = p["phi_e"][0]["W"]
    wa, wb, wc, wd = W[0:D], W[D:2 * D], W[2 * D:3 * D], W[3 * D:4 * D]
    bp = p["phi_e"][0]["b"][None]
    v2, c2 = p["phi_e"][1]["W"], p["phi_e"][1]["b"][None]
    v3, c3 = p["phi_e"][2]["W"], p["phi_e"][2]["b"][None]
    din = ea.shape[1]
    n_out = 2 if skip_out else 1
    ws = (w1, b1, w2, b2, wa, wb, wc, wd, bp, v2, c2, v3, c3)
    outs = pl.pallas_call(
        _edge_kernel_skip if skip_out else _edge_kernel_noskip,
        grid=(N_E // _R,),
        in_specs=[pl.BlockSpec((_R, din), lambda i: (i, 0))]
        + [pl.BlockSpec((_R, D), lambda i: (i, 0))] * 3
        + [_full(w) for w in ws],
        out_specs=[pl.BlockSpec((_R, D), lambda i: (i, 0))] * n_out,
        out_shape=[jax.ShapeDtypeStruct((N_E, D), jnp.float32)] * n_out,
    )(ea, xvs, xvd, xub, *ws)
    return outs if skip_out else (outs[0], outs[0])


def _phi_v_kernel_skip(xv, evp, degp, xun, xin, wa, wb, wc, bp, v2, c2, v3, c3,
                       o_pre, o_post):
    r = 1.0 / jnp.maximum(degp[0, :, :] + degp[1, :, :], 1.0)
    etov = (evp[0] + evp[1]) * r
    h = _sp(xv[...] @ wa[...] + etov @ wb[...] + xun[...] @ wc[...] + bp[...])
    h = _sp(h @ v2[...] + c2[...])
    v_pre = _sp(h @ v3[...] + c3[...])
    o_pre[...] = v_pre
    o_post[...] = v_pre + xin[...]


def _phi_v_kernel_noskip(xv, evp, degp, xun, wa, wb, wc, bp, v2, c2, v3, c3,
                         o_pre):
    r = 1.0 / jnp.maximum(degp[0, :, :] + degp[1, :, :], 1.0)
    etov = (evp[0] + evp[1]) * r
    h = _sp(xv[...] @ wa[...] + etov @ wb[...] + xun[...] @ wc[...] + bp[...])
    h = _sp(h @ v2[...] + c2[...])
    o_pre[...] = _sp(h @ v3[...] + c3[...])


def _phi_v(xv, evp, degp, xun, xin, p, skip_out):
    W = p["phi_v"][0]["W"]
    wa, wb, wc = W[0:D], W[D:2 * D], W[2 * D:3 * D]
    bp = p["phi_v"][0]["b"][None]
    v2, c2 = p["phi_v"][1]["W"], p["phi_v"][1]["b"][None]
    v3, c3 = p["phi_v"][2]["W"], p["phi_v"][2]["b"][None]
    degp3 = degp[:, :, None]
    ws = (wa, wb, wc, bp, v2, c2, v3, c3)
    n_out = 2 if skip_out else 1
    row = pl.BlockSpec((_R, D), lambda i: (i, 0))
    ins = [xv, evp, degp3, xun] + ([xin] if skip_out else [])
    in_specs = [row,
                pl.BlockSpec((2, _R, D), lambda i: (0, i, 0)),
                pl.BlockSpec((2, _R, 1), lambda i: (0, i, 0)),
                row] + ([row] if skip_out else [])
    outs = pl.pallas_call(
        _phi_v_kernel_skip if skip_out else _phi_v_kernel_noskip,
        grid=(N_V // _R,),
        in_specs=in_specs + [_full(w) for w in ws],
        out_specs=[row] * n_out,
        out_shape=[jax.ShapeDtypeStruct((N_V, D), jnp.float32)] * n_out,
    )(*ins, *ws)
    return outs if skip_out else (outs[0], outs[0])


def _phi_u_kernel(uep, uvp, cep, cvp, xu, uin, wa, wb, wc, bp, v2, c2, v3, c3,
                  o_ref):
    ue = (uep[0] + uep[1]) / jnp.maximum(cep[0] + cep[1], 1.0)
    uv = (uvp[0] + uvp[1]) / jnp.maximum(cvp[0] + cvp[1], 1.0)
    h = _sp(ue @ wa[...] + uv @ wb[...] + xu[...] @ wc[...] + bp[...])
    h = _sp(h @ v2[...] + c2[...])
    u_pre = _sp(h @ v3[...] + c3[...])
    o_ref[...] = u_pre + uin[...]


def _phi_u(uep, uvp, cep, cvp, xu, uin, p):
    W = p["phi_u"][0]["W"]
    wa, wb, wc = W[0:D], W[D:2 * D], W[2 * D:3 * D]
    bp = p["phi_u"][0]["b"][None]
    v2, c2 = p["phi_u"][1]["W"], p["phi_u"][1]["b"][None]
    v3, c3 = p["phi_u"][2]["W"], p["phi_u"][2]["b"][None]
    return pl.pallas_call(
        _phi_u_kernel,
        out_shape=jax.ShapeDtypeStruct((N_G, D), jnp.float32),
    )(uep, uvp, cep[:, :, None], cvp[:, :, None], xu, uin,
      wa, wb, wc, bp, v2, c2, v3, c3)


def _head_kernel(uv1, uv2, uv3, ue1, ue2, ue3, cep, cvp, uu3,
                 w0, b0, w1, b1, w2, b2, o_ref):
    mv_sum = uv1[0] + uv1[1] + uv2[0] + uv2[1] + uv3[0] + uv3[1]
    me_sum = ue1[0] + ue1[1] + ue2[0] + ue2[1] + ue3[0] + ue3[1]
    mv = mv_sum / jnp.maximum(cvp[0] + cvp[1], 1.0)
    me = me_sum / jnp.maximum(cep[0] + cep[1], 1.0)
    z = jnp.zeros_like(mv)
    tmp = jnp.concatenate([z, mv, z, me, uu3[...]], axis=1)
    h = _sp(tmp @ w0[...] + b0[...])
    h = _sp(h @ w1[...] + b1[...])
    o_ref[...] = h @ w2[...] + b2[...]


def _module(x, edge_attr, state_feat, uin, idx2d, counts, p, skip):
    src2d, dst2d, bb2d, nb2d, zeros = idx2d
    degp, cep, cvp = counts
    xv = _pre_v(x, p["pre_v"])
    xu = _pre_u(state_feat, p["pre_u"])
    xvs, xvd, xub, xun = _sc_gather(xv, xu, src2d, dst2d, bb2d, nb2d)
    e_pre, e_post = _edge_mlp(edge_attr, xvs, xvd, xub, p, skip)
    evp, uep = _sc_scatter_edges(e_pre, dst2d, bb2d, zeros)
    v_pre, x_post = _phi_v(xv, evp, degp, xun, x, p, skip)
    uvp = _sc_scatter_nodes(v_pre, nb2d, zeros)
    uu = _phi_u(uep, uvp, cep, cvp, xu, uin, p)
    return x_post, e_post, uu, uvp, uep


def kernel(x, edge_index, edge_attr, state, batch, bond_batch, params):
    src2d = edge_index[0].reshape(_E_CHUNKS * _E_K, _E_W)
    dst2d = edge_index[1].reshape(_E_CHUNKS * _E_K, _E_W)
    bb2d = bond_batch.reshape(_E_CHUNKS * _E_K, _E_W)
    nb2d = batch.reshape(_N_CHUNKS * _N_K, _N_W)
    zeros = jnp.zeros((N_V, D), jnp.float32)
    zeros1 = jnp.zeros((N_V,), jnp.float32)
    zg = jnp.zeros((N_G, D), jnp.float32)

    degp, cep, cvp = _sc_counts(dst2d, bb2d, nb2d, zeros1)
    counts = (degp, cep, cvp)
    idx2d = (src2d, dst2d, bb2d, nb2d, zeros)

    x1, ee1, uu1, uvp1, uep1 = _module(x, edge_attr, state, zg, idx2d,
                                       counts, params["m1"], False)
    x2, ee2, uu2, uvp2, uep2 = _module(x1, ee1, uu1, uu1, idx2d,
                                       counts, params["m2"], True)
    _, _, uu3, uvp3, uep3 = _module(x2, ee2, uu2, uu2, idx2d,
                                    counts, params["m3"], False)

    hl = params["hiddens"]
    out = pl.pallas_call(
        _head_kernel,
        out_shape=jax.ShapeDtypeStruct((N_G, 1), jnp.float32),
    )(uvp1, uvp2, uvp3, uep1, uep2, uep3, cep[:, :, None], cvp[:, :, None],
      uu3, hl[0]["W"], hl[0]["b"][None], hl[1]["W"], hl[1]["b"][None],
      hl[2]["W"], hl[2]["b"][None])
    return out


# speedup vs baseline: 1.0876x; 1.0876x over previous
import functools

import jax
import jax.numpy as jnp
from jax import lax
from jax.experimental import pallas as pl
from jax.experimental.pallas import tpu as pltpu
from jax.experimental.pallas import tpu_sc as plsc

N_V = 50000
N_E = 800000
N_G = 512
D = 32

_E_W = 100
_E_K = 8
_E_ROWS = _E_W * _E_K
_E_CHUNKS = N_E // _E_ROWS
_N_W = 25
_N_K = 8
_N_ROWS = _N_W * _N_K
_N_CHUNKS = N_V // _N_ROWS

_NW = 32
_E_TRIPS = (_E_CHUNKS + _NW - 1) // _NW
_N_TRIPS = (_N_CHUNKS + _NW - 1) // _NW

_MESH = plsc.VectorSubcoreMesh(core_axis_name="c", subcore_axis_name="s")
_NSUB = 16
_VSUB = 10
_VROWS = N_V // _VSUB
_GROWS = N_G // _NSUB


def _worker_id():
    return lax.axis_index("s") * 2 + lax.axis_index("c")


@functools.partial(
    pl.kernel,
    out_type=(
        jax.ShapeDtypeStruct((N_E, D), jnp.float32),
        jax.ShapeDtypeStruct((N_E, D), jnp.float32),
        jax.ShapeDtypeStruct((N_E, D), jnp.float32),
        jax.ShapeDtypeStruct((N_V, D), jnp.float32),
    ),
    mesh=_MESH,
    compiler_params=pltpu.CompilerParams(use_tc_tiling_on_sc=False),
    scratch_types=[
        pltpu.VMEM_SHARED((N_V, D), jnp.float32),
        pltpu.VMEM_SHARED((N_G, D), jnp.float32),
        pltpu.VMEM((_E_K, _E_W), jnp.int32),
        pltpu.VMEM((_E_K, _E_W), jnp.int32),
        pltpu.VMEM((_E_K, _E_W), jnp.int32),
        pltpu.VMEM((2, _E_W, D), jnp.float32),
        pltpu.VMEM((2, _E_W, D), jnp.float32),
        pltpu.VMEM((2, _E_W, D), jnp.float32),
        pltpu.VMEM((_N_K, _N_W), jnp.int32),
        pltpu.VMEM((2, _N_W, D), jnp.float32),
        pltpu.SemaphoreType.DMA,
        pltpu.SemaphoreType.DMA,
    ],
)
def _sc_gather(xv_hbm, xu_hbm, src_hbm, dst_hbm, bb_hbm, nb_hbm,
               os_hbm, od_hbm, ob_hbm, on_hbm,
               xv_spm, xu_spm, si, di, bi, gs, gd, gb, ni, gn, gsem, wsem):
    sid = lax.axis_index("s")
    wid = _worker_id()

    @pl.when(sid < _VSUB)
    def _():
        pltpu.sync_copy(xv_hbm.at[pl.ds(sid * _VROWS, _VROWS)],
                        xv_spm.at[pl.ds(sid * _VROWS, _VROWS)])

    pltpu.sync_copy(xu_hbm.at[pl.ds(sid * _GROWS, _GROWS)],
                    xu_spm.at[pl.ds(sid * _GROWS, _GROWS)])
    plsc.subcore_barrier()

    @pl.loop(0, _E_TRIPS)
    def _(t):
        c = wid + t * _NW

        @pl.when(c < _E_CHUNKS)
        def _():
            base = c * _E_ROWS
            pltpu.sync_copy(src_hbm.at[pl.ds(c * _E_K, _E_K)], si)
            pltpu.sync_copy(dst_hbm.at[pl.ds(c * _E_K, _E_K)], di)
            pltpu.sync_copy(bb_hbm.at[pl.ds(c * _E_K, _E_K)], bi)

            w = {}
            for j in range(_E_K):
                b = j & 1
                ww = pl.ds(base + j * _E_W, _E_W)
                if j >= 2:
                    for h in w.pop(j - 2):
                        h.wait()
                pltpu.async_copy(xv_spm.at[si.at[j]], gs.at[b], gsem).wait()
                pltpu.async_copy(xv_spm.at[di.at[j]], gd.at[b], gsem).wait()
                pltpu.async_copy(xu_spm.at[bi.at[j]], gb.at[b], gsem).wait()
                w[j] = (
                    pltpu.async_copy(gs.at[b], os_hbm.at[ww], wsem),
                    pltpu.async_copy(gd.at[b], od_hbm.at[ww], wsem),
                    pltpu.async_copy(gb.at[b], ob_hbm.at[ww], wsem),
                )
            for j in (_E_K - 2, _E_K - 1):
                for h in w.pop(j):
                    h.wait()

    @pl.loop(0, _N_TRIPS)
    def _(t):
        c = wid + t * _NW

        @pl.when(c < _N_CHUNKS)
        def _():
            base = c * _N_ROWS
            pltpu.sync_copy(nb_hbm.at[pl.ds(c * _N_K, _N_K)], ni)

            w = {}
            for j in range(_N_K):
                if j >= 2:
                    w.pop(j - 2).wait()
                pltpu.async_copy(xu_spm.at[ni.at[j]], gn.at[j & 1],
                                 gsem).wait()
                w[j] = pltpu.async_copy(
                    gn.at[j & 1], on_hbm.at[pl.ds(base + j * _N_W, _N_W)],
                    wsem)
            for j in (_N_K - 2, _N_K - 1):
                w.pop(j).wait()


@functools.partial(
    pl.kernel,
    out_type=(
        jax.ShapeDtypeStruct((2, N_V, D), jnp.float32),
        jax.ShapeDtypeStruct((2, N_G, D), jnp.float32),
    ),
    mesh=_MESH,
    compiler_params=pltpu.CompilerParams(use_tc_tiling_on_sc=False),
    scratch_types=[
        pltpu.VMEM_SHARED((N_V, D), jnp.float32),
        pltpu.VMEM_SHARED((N_G, D), jnp.float32),
        pltpu.VMEM((_E_K, _E_W), jnp.int32),
        pltpu.VMEM((_E_K, _E_W), jnp.int32),
        pltpu.VMEM((4, _E_W, D), jnp.float32),
        pltpu.SemaphoreType.DMA,
        pltpu.SemaphoreType.DMA,
    ],
)
def _sc_scatter_edges(e_hbm, dst_hbm, bb_hbm, z_hbm, ov_hbm, ou_hbm,
                      accv, accu, di, bi, ge, lsem, ssem):
    cid = lax.axis_index("c")
    sid = lax.axis_index("s")
    wid = _worker_id()

    @pl.when(sid < _VSUB)
    def _():
        pltpu.sync_copy(z_hbm.at[pl.ds(sid * _VROWS, _VROWS)],
                        accv.at[pl.ds(sid * _VROWS, _VROWS)])

    pltpu.sync_copy(z_hbm.at[pl.ds(sid * _GROWS, _GROWS)],
                    accu.at[pl.ds(sid * _GROWS, _GROWS)])
    plsc.subcore_barrier()

    @pl.loop(0, _E_TRIPS)
    def _(t):
        c = wid + t * _NW

        @pl.when(c < _E_CHUNKS)
        def _():
            base = c * _E_ROWS
            pltpu.sync_copy(dst_hbm.at[pl.ds(c * _E_K, _E_K)], di)
            pltpu.sync_copy(bb_hbm.at[pl.ds(c * _E_K, _E_K)], bi)

            def load(j):
                return pltpu.async_copy(
                    e_hbm.at[pl.ds(base + j * _E_W, _E_W)], ge.at[j & 3],
                    lsem)

            ld = {0: load(0), 1: load(1)}
            for j in range(_E_K):
                b = j & 3
                ld.pop(j).wait()
                if j + 2 < _E_K:
                    ld[j + 2] = load(j + 2)
                pltpu.async_copy(ge.at[b], accv.at[di.at[j]], ssem,
                                 add=True).wait()
                pltpu.async_copy(ge.at[b], accu.at[bi.at[j]], ssem,
                                 add=True).wait()

    plsc.subcore_barrier()

    @pl.when(sid < _VSUB)
    def _():
        pltpu.sync_copy(accv.at[pl.ds(sid * _VROWS, _VROWS)],
                        ov_hbm.at[cid, pl.ds(sid * _VROWS, _VROWS)])

    pltpu.sync_copy(accu.at[pl.ds(sid * _GROWS, _GROWS)],
                    ou_hbm.at[cid, pl.ds(sid * _GROWS, _GROWS)])


@functools.partial(
    pl.kernel,
    out_type=jax.ShapeDtypeStruct((2, N_G, D), jnp.float32),
    mesh=_MESH,
    compiler_params=pltpu.CompilerParams(use_tc_tiling_on_sc=False),
    scratch_types=[
        pltpu.VMEM_SHARED((N_G, D), jnp.float32),
        pltpu.VMEM((_N_K, _N_W), jnp.int32),
        pltpu.VMEM((4, _N_W, D), jnp.float32),
        pltpu.SemaphoreType.DMA,
        pltpu.SemaphoreType.DMA,
    ],
)
def _sc_scatter_nodes(v_hbm, nb_hbm, z_hbm, ou_hbm, accu, ni, gv, lsem, ssem):
    cid = lax.axis_index("c")
    sid = lax.axis_index("s")
    wid = _worker_id()
    pltpu.sync_copy(z_hbm.at[pl.ds(sid * _GROWS, _GROWS)],
                    accu.at[pl.ds(sid * _GROWS, _GROWS)])
    plsc.subcore_barrier()

    @pl.loop(0, _N_TRIPS)
    def _(t):
        c = wid + t * _NW

        @pl.when(c < _N_CHUNKS)
        def _():
            base = c * _N_ROWS
            pltpu.sync_copy(nb_hbm.at[pl.ds(c * _N_K, _N_K)], ni)

            def load(j):
                return pltpu.async_copy(
                    v_hbm.at[pl.ds(base + j * _N_W, _N_W)], gv.at[j & 3],
                    lsem)

            ld = {0: load(0), 1: load(1)}
            for j in range(_N_K):
                ld.pop(j).wait()
                if j + 2 < _N_K:
                    ld[j + 2] = load(j + 2)
                pltpu.async_copy(gv.at[j & 3], accu.at[ni.at[j]], ssem,
                                 add=True).wait()

    plsc.subcore_barrier()
    pltpu.sync_copy(accu.at[pl.ds(sid * _GROWS, _GROWS)],
                    ou_hbm.at[cid, pl.ds(sid * _GROWS, _GROWS)])


@functools.partial(
    pl.kernel,
    out_type=(
        jax.ShapeDtypeStruct((2, N_V), jnp.float32),
        jax.ShapeDtypeStruct((2, N_G), jnp.float32),
        jax.ShapeDtypeStruct((2, N_G), jnp.float32),
    ),
    mesh=_MESH,
    compiler_params=pltpu.CompilerParams(use_tc_tiling_on_sc=False),
    scratch_types=[
        pltpu.VMEM_SHARED((N_V,), jnp.float32),
        pltpu.VMEM_SHARED((N_G,), jnp.float32),
        pltpu.VMEM_SHARED((N_G,), jnp.float32),
        pltpu.VMEM((_E_K, _E_W), jnp.int32),
        pltpu.VMEM((_E_K, _E_W), jnp.int32),
        pltpu.VMEM((_N_K, _N_W), jnp.int32),
        pltpu.VMEM((128,), jnp.float32),
        pltpu.SemaphoreType.DMA,
    ],
)
def _sc_counts(dst_hbm, bb_hbm, nb_hbm, z_hbm, od_hbm, oe_hbm, ov_hbm,
               accd, acce, accv, di, bi, ni, ones, ssem):
    cid = lax.axis_index("c")
    sid = lax.axis_index("s")
    wid = _worker_id()
    for j in range(8):
        ones[pl.ds(j * 16, 16)] = jnp.ones((16,), jnp.float32)

    @pl.when(sid < _VSUB)
    def _():
        pltpu.sync_copy(z_hbm.at[pl.ds(sid * _VROWS, _VROWS)],
                        accd.at[pl.ds(sid * _VROWS, _VROWS)])

    pltpu.sync_copy(z_hbm.at[pl.ds(sid * _GROWS, _GROWS)],
                    acce.at[pl.ds(sid * _GROWS, _GROWS)])
    pltpu.sync_copy(z_hbm.at[pl.ds(sid * _GROWS, _GROWS)],
                    accv.at[pl.ds(sid * _GROWS, _GROWS)])
    plsc.subcore_barrier()

    @pl.loop(0, _E_TRIPS)
    def _(t):
        c = wid + t * _NW

        @pl.when(c < _E_CHUNKS)
        def _():
            pltpu.sync_copy(dst_hbm.at[pl.ds(c * _E_K, _E_K)], di)
            pltpu.sync_copy(bb_hbm.at[pl.ds(c * _E_K, _E_K)], bi)
            for j in range(_E_K):
                pltpu.async_copy(ones.at[pl.ds(0, _E_W)], accd.at[di.at[j]],
                                 ssem, add=True).wait()
                pltpu.async_copy(ones.at[pl.ds(0, _E_W)], acce.at[bi.at[j]],
                                 ssem, add=True).wait()

    @pl.loop(0, _N_TRIPS)
    def _(t):
        c = wid + t * _NW

        @pl.when(c < _N_CHUNKS)
        def _():
            pltpu.sync_copy(nb_hbm.at[pl.ds(c * _N_K, _N_K)], ni)
            for j in range(_N_K):
                pltpu.async_copy(ones.at[pl.ds(0, _N_W)], accv.at[ni.at[j]],
                                 ssem, add=True).wait()

    plsc.subcore_barrier()

    @pl.when(sid < _VSUB)
    def _():
        pltpu.sync_copy(accd.at[pl.ds(sid * _VROWS, _VROWS)],
                        od_hbm.at[cid, pl.ds(sid * _VROWS, _VROWS)])

    pltpu.sync_copy(acce.at[pl.ds(sid * _GROWS, _GROWS)],
                    oe_hbm.at[cid, pl.ds(sid * _GROWS, _GROWS)])
    pltpu.sync_copy(accv.at[pl.ds(sid * _GROWS, _GROWS)],
                    ov_hbm.at[cid, pl.ds(sid * _GROWS, _GROWS)])


_R = 2000


def _full(a):
    if a.ndim == 3:
        return pl.BlockSpec(a.shape, lambda i: (0, 0, 0))
    if a.ndim == 2:
        return pl.BlockSpec(a.shape, lambda i: (0, 0))
    return pl.BlockSpec(a.shape, lambda i: (0,))


def _sp(x):
    return jax.nn.softplus(x)


def _pre_v_kernel(x_ref, w1, b1, w2, b2, o_ref):
    h = _sp(x_ref[...] @ w1[...] + b1[...])
    o_ref[...] = _sp(h @ w2[...] + b2[...])


def _pre_v(x, p):
    w1, b1 = p[0]["W"], p[0]["b"][None]
    w2, b2 = p[1]["W"], p[1]["b"][None]
    din = x.shape[1]
    return pl.pallas_call(
        _pre_v_kernel,
        grid=(N_V // _R,),
        in_specs=[pl.BlockSpec((_R, din), lambda i: (i, 0))]
        + [_full(w) for w in (w1, b1, w2, b2)],
        out_specs=pl.BlockSpec((_R, D), lambda i: (i, 0)),
        out_shape=jax.ShapeDtypeStruct((N_V, D), jnp.float32),
    )(x, w1, b1, w2, b2)


def _pre_u_kernel(u_ref, w1, b1, w2, b2, o_ref):
    h = _sp(u_ref[...] @ w1[...] + b1[...])
    o_ref[...] = _sp(h @ w2[...] + b2[...])


def _pre_u(u, p):
    w1, b1 = p[0]["W"], p[0]["b"][None]
    w2, b2 = p[1]["W"], p[1]["b"][None]
    return pl.pallas_call(
        _pre_u_kernel,
        out_shape=jax.ShapeDtypeStruct((N_G, D), jnp.float32),
    )(u, w1, b1, w2, b2)


def _edge_body(ea, xvs, xvd, xub, w1, b1, w2, b2, wa, wb, wc, wd, bp, v2, c2,
               v3, c3):
    xe = _sp(ea[...] @ w1[...] + b1[...])
    xe = _sp(xe @ w2[...] + b2[...])
    h = _sp(xvs[...] @ wa[...] + xvd[...] @ wb[...] + xe @ wc[...]
            + xub[...] @ wd[...] + bp[...])
    h = _sp(h @ v2[...] + c2[...])
    return _sp(h @ v3[...] + c3[...])


def _edge_kernel_skip(ea, xvs, xvd, xub, w1, b1, w2, b2, wa, wb, wc, wd, bp,
                      v2, c2, v3, c3, o_pre, o_post):
    e_pre = _edge_body(ea, xvs, xvd, xub, w1, b1, w2, b2, wa, wb, wc, wd, bp,
                       v2, c2, v3, c3)
    o_pre[...] = e_pre
    o_post[...] = e_pre + ea[...]


def _edge_kernel_noskip(ea, xvs, xvd, xub, w1, b1, w2, b2, wa, wb, wc, wd, bp,
                        v2, c2, v3, c3, o_pre):
    o_pre[...] = _edge_body(ea, xvs, xvd, xub, w1, b1, w2, b2, wa, wb, wc, wd,
                            bp, v2, c2, v3, c3)


def _edge_mlp(ea, xvs, xvd, xub, p, skip_out):
    w1, b1 = p["pre_e"][0]["W"], p["pre_e"][0]["b"][None]
    w2, b2 = p["pre_e"][1]["W"], p["pre_e"][1]["b"][None]
    W = p["phi_e"][0]["W"]
    wa, wb, wc, wd = W[0:D], W[D:2 * D], W[2 * D:3 * D], W[3 * D:4 * D]
    bp = p["phi_e"][0]["b"][None]
    v2, c2 = p["phi_e"][1]["W"], p["phi_e"][1]["b"][None]
    v3, c3 = p["phi_e"][2]["W"], p["phi_e"][2]["b"][None]
    din = ea.shape[1]
    n_out = 2 if skip_out else 1
    ws = (w1, b1, w2, b2, wa, wb, wc, wd, bp, v2, c2, v3, c3)
    outs = pl.pallas_call(
        _edge_kernel_skip if skip_out else _edge_kernel_noskip,
        grid=(N_E // _R,),
        in_specs=[pl.BlockSpec((_R, din), lambda i: (i, 0))]
        + [pl.BlockSpec((_R, D), lambda i: (i, 0))] * 3
        + [_full(w) for w in ws],
        out_specs=[pl.BlockSpec((_R, D), lambda i: (i, 0))] * n_out,
        out_shape=[jax.ShapeDtypeStruct((N_E, D), jnp.float32)] * n_out,
    )(ea, xvs, xvd, xub, *ws)
    return outs if skip_out else (outs[0], outs[0])


def _phi_v_kernel_skip(xv, evp, degp, xun, xin, wa, wb, wc, bp, v2, c2, v3, c3,
                       o_pre, o_post):
    r = 1.0 / jnp.maximum(degp[0, :, :] + degp[1, :, :], 1.0)
    etov = (evp[0] + evp[1]) * r
    h = _sp(xv[...] @ wa[...] + etov @ wb[...] + xun[...] @ wc[...] + bp[...])
    h = _sp(h @ v2[...] + c2[...])
    v_pre = _sp(h @ v3[...] + c3[...])
    o_pre[...] = v_pre
    o_post[...] = v_pre + xin[...]


def _phi_v_kernel_noskip(xv, evp, degp, xun, wa, wb, wc, bp, v2, c2, v3, c3,
                         o_pre):
    r = 1.0 / jnp.maximum(degp[0, :, :] + degp[1, :, :], 1.0)
    etov = (evp[0] + evp[1]) * r
    h = _sp(xv[...] @ wa[...] + etov @ wb[...] + xun[...] @ wc[...] + bp[...])
    h = _sp(h @ v2[...] + c2[...])
    o_pre[...] = _sp(h @ v3[...] + c3[...])


def _phi_v(xv, evp, degp, xun, xin, p, skip_out):
    W = p["phi_v"][0]["W"]
    wa, wb, wc = W[0:D], W[D:2 * D], W[2 * D:3 * D]
    bp = p["phi_v"][0]["b"][None]
    v2, c2 = p["phi_v"][1]["W"], p["phi_v"][1]["b"][None]
    v3, c3 = p["phi_v"][2]["W"], p["phi_v"][2]["b"][None]
    degp3 = degp[:, :, None]
    ws = (wa, wb, wc, bp, v2, c2, v3, c3)
    n_out = 2 if skip_out else 1
    row = pl.BlockSpec((_R, D), lambda i: (i, 0))
    ins = [xv, evp, degp3, xun] + ([xin] if skip_out else [])
    in_specs = [row,
                pl.BlockSpec((2, _R, D), lambda i: (0, i, 0)),
                pl.BlockSpec((2, _R, 1), lambda i: (0, i, 0)),
                row] + ([row] if skip_out else [])
    outs = pl.pallas_call(
        _phi_v_kernel_skip if skip_out else _phi_v_kernel_noskip,
        grid=(N_V // _R,),
        in_specs=in_specs + [_full(w) for w in ws],
        out_specs=[row] * n_out,
        out_shape=[jax.ShapeDtypeStruct((N_V, D), jnp.float32)] * n_out,
    )(*ins, *ws)
    return outs if skip_out else (outs[0], outs[0])


def _phi_u_kernel(uep, uvp, cep, cvp, xu, uin, wa, wb, wc, bp, v2, c2, v3, c3,
                  o_ref):
    ue = (uep[0] + uep[1]) / jnp.maximum(cep[0] + cep[1], 1.0)
    uv = (uvp[0] + uvp[1]) / jnp.maximum(cvp[0] + cvp[1], 1.0)
    h = _sp(ue @ wa[...] + uv @ wb[...] + xu[...] @ wc[...] + bp[...])
    h = _sp(h @ v2[...] + c2[...])
    u_pre = _sp(h @ v3[...] + c3[...])
    o_ref[...] = u_pre + uin[...]


def _phi_u(uep, uvp, cep, cvp, xu, uin, p):
    W = p["phi_u"][0]["W"]
    wa, wb, wc = W[0:D], W[D:2 * D], W[2 * D:3 * D]
    bp = p["phi_u"][0]["b"][None]
    v2, c2 = p["phi_u"][1]["W"], p["phi_u"][1]["b"][None]
    v3, c3 = p["phi_u"][2]["W"], p["phi_u"][2]["b"][None]
    return pl.pallas_call(
        _phi_u_kernel,
        out_shape=jax.ShapeDtypeStruct((N_G, D), jnp.float32),
    )(uep, uvp, cep[:, :, None], cvp[:, :, None], xu, uin,
      wa, wb, wc, bp, v2, c2, v3, c3)


def _head_kernel(uv1, uv2, uv3, ue1, ue2, ue3, cep, cvp, uu3,
                 w0, b0, w1, b1, w2, b2, o_ref):
    mv_sum = uv1[0] + uv1[1] + uv2[0] + uv2[1] + uv3[0] + uv3[1]
    me_sum = ue1[0] + ue1[1] + ue2[0] + ue2[1] + ue3[0] + ue3[1]
    mv = mv_sum / jnp.maximum(cvp[0] + cvp[1], 1.0)
    me = me_sum / jnp.maximum(cep[0] + cep[1], 1.0)
    z = jnp.zeros_like(mv)
    tmp = jnp.concatenate([z, mv, z, me, uu3[...]], axis=1)
    h = _sp(tmp @ w0[...] + b0[...])
    h = _sp(h @ w1[...] + b1[...])
    o_ref[...] = h @ w2[...] + b2[...]


def _module(x, edge_attr, state_feat, uin, idx2d, counts, p, skip):
    src2d, dst2d, bb2d, nb2d, zeros = idx2d
    degp, cep, cvp = counts
    xv = _pre_v(x, p["pre_v"])
    xu = _pre_u(state_feat, p["pre_u"])
    xvs, xvd, xub, xun = _sc_gather(xv, xu, src2d, dst2d, bb2d, nb2d)
    e_pre, e_post = _edge_mlp(edge_attr, xvs, xvd, xub, p, skip)
    evp, uep = _sc_scatter_edges(e_pre, dst2d, bb2d, zeros)
    v_pre, x_post = _phi_v(xv, evp, degp, xun, x, p, skip)
    uvp = _sc_scatter_nodes(v_pre, nb2d, zeros)
    uu = _phi_u(uep, uvp, cep, cvp, xu, uin, p)
    return x_post, e_post, uu, uvp, uep


def kernel(x, edge_index, edge_attr, state, batch, bond_batch, params):
    src2d = edge_index[0].reshape(_E_CHUNKS * _E_K, _E_W)
    dst2d = edge_index[1].reshape(_E_CHUNKS * _E_K, _E_W)
    bb2d = bond_batch.reshape(_E_CHUNKS * _E_K, _E_W)
    nb2d = batch.reshape(_N_CHUNKS * _N_K, _N_W)
    zeros = jnp.zeros((N_V, D), jnp.float32)
    zeros1 = jnp.zeros((N_V,), jnp.float32)
    zg = jnp.zeros((N_G, D), jnp.float32)

    degp, cep, cvp = _sc_counts(dst2d, bb2d, nb2d, zeros1)
    counts = (degp, cep, cvp)
    idx2d = (src2d, dst2d, bb2d, nb2d, zeros)

    x1, ee1, uu1, uvp1, uep1 = _module(x, edge_attr, state, zg, idx2d,
                                       counts, params["m1"], False)
    x2, ee2, uu2, uvp2, uep2 = _module(x1, ee1, uu1, uu1, idx2d,
                                       counts, params["m2"], True)
    _, _, uu3, uvp3, uep3 = _module(x2, ee2, uu2, uu2, idx2d,
                                    counts, params["m3"], False)

    hl = params["hiddens"]
    out = pl.pallas_call(
        _head_kernel,
        out_shape=jax.ShapeDtypeStruct((N_G, 1), jnp.float32),
    )(uvp1, uvp2, uvp3, uep1, uep2, uep3, cep[:, :, None], cvp[:, :, None],
      uu3, hl[0]["W"], hl[0]["b"][None], hl[1]["W"], hl[1]["b"][None],
      hl[2]["W"], hl[2]["b"][None])
    return out


# speedup vs baseline: 1.1000x; 1.0114x over previous
import functools

import jax
import jax.numpy as jnp
from jax import lax
from jax.experimental import pallas as pl
from jax.experimental.pallas import tpu as pltpu
from jax.experimental.pallas import tpu_sc as plsc

N_V = 50000
N_E = 800000
N_G = 512
D = 32

_E_W = 100
_E_K = 8
_E_ROWS = _E_W * _E_K
_E_CHUNKS = N_E // _E_ROWS
_N_W = 25
_N_K = 8
_N_ROWS = _N_W * _N_K
_N_CHUNKS = N_V // _N_ROWS

_NW = 32
_E_TRIPS = (_E_CHUNKS + _NW - 1) // _NW
_N_TRIPS = (_N_CHUNKS + _NW - 1) // _NW

_MESH = plsc.VectorSubcoreMesh(core_axis_name="c", subcore_axis_name="s")
_NSUB = 16
_VSUB = 10
_VROWS = N_V // _VSUB
_GROWS = N_G // _NSUB


def _worker_id():
    return lax.axis_index("s") * 2 + lax.axis_index("c")


@functools.partial(
    pl.kernel,
    out_type=(
        jax.ShapeDtypeStruct((N_E, D), jnp.float32),
        jax.ShapeDtypeStruct((N_E, D), jnp.float32),
        jax.ShapeDtypeStruct((N_E, D), jnp.float32),
        jax.ShapeDtypeStruct((N_V, D), jnp.float32),
    ),
    mesh=_MESH,
    compiler_params=pltpu.CompilerParams(use_tc_tiling_on_sc=False),
    scratch_types=[
        pltpu.VMEM_SHARED((N_V, D), jnp.float32),
        pltpu.VMEM_SHARED((N_G, D), jnp.float32),
        pltpu.VMEM((_E_K, _E_W), jnp.int32),
        pltpu.VMEM((_E_K, _E_W), jnp.int32),
        pltpu.VMEM((_E_K, _E_W), jnp.int32),
        pltpu.VMEM((2, _E_W, D), jnp.float32),
        pltpu.VMEM((2, _E_W, D), jnp.float32),
        pltpu.VMEM((2, _E_W, D), jnp.float32),
        pltpu.VMEM((_N_K, _N_W), jnp.int32),
        pltpu.VMEM((2, _N_W, D), jnp.float32),
        pltpu.SemaphoreType.DMA,
        pltpu.SemaphoreType.DMA,
    ],
)
def _sc_gather(xv_hbm, xu_hbm, src_hbm, dst_hbm, bb_hbm, nb_hbm,
               os_hbm, od_hbm, ob_hbm, on_hbm,
               xv_spm, xu_spm, si, di, bi, gs, gd, gb, ni, gn, gsem, wsem):
    sid = lax.axis_index("s")
    wid = _worker_id()

    @pl.when(sid < _VSUB)
    def _():
        pltpu.sync_copy(xv_hbm.at[pl.ds(sid * _VROWS, _VROWS)],
                        xv_spm.at[pl.ds(sid * _VROWS, _VROWS)])

    pltpu.sync_copy(xu_hbm.at[pl.ds(sid * _GROWS, _GROWS)],
                    xu_spm.at[pl.ds(sid * _GROWS, _GROWS)])
    plsc.subcore_barrier()

    @pl.loop(0, _E_TRIPS)
    def _(t):
        c = wid + t * _NW

        @pl.when(c < _E_CHUNKS)
        def _():
            base = c * _E_ROWS
            pltpu.sync_copy(src_hbm.at[pl.ds(c * _E_K, _E_K)], si)
            pltpu.sync_copy(dst_hbm.at[pl.ds(c * _E_K, _E_K)], di)
            pltpu.sync_copy(bb_hbm.at[pl.ds(c * _E_K, _E_K)], bi)

            w = {}
            for j in range(_E_K):
                b = j & 1
                ww = pl.ds(base + j * _E_W, _E_W)
                if j >= 2:
                    for h in w.pop(j - 2):
                        h.wait()
                h1 = pltpu.async_copy(xv_spm.at[si.at[j]], gs.at[b], gsem)
                h2 = pltpu.async_copy(xv_spm.at[di.at[j]], gd.at[b], gsem)
                h3 = pltpu.async_copy(xu_spm.at[bi.at[j]], gb.at[b], gsem)
                h1.wait()
                h2.wait()
                h3.wait()
                w[j] = (
                    pltpu.async_copy(gs.at[b], os_hbm.at[ww], wsem),
                    pltpu.async_copy(gd.at[b], od_hbm.at[ww], wsem),
                    pltpu.async_copy(gb.at[b], ob_hbm.at[ww], wsem),
                )
            for j in (_E_K - 2, _E_K - 1):
                for h in w.pop(j):
                    h.wait()

    @pl.loop(0, _N_TRIPS)
    def _(t):
        c = wid + t * _NW

        @pl.when(c < _N_CHUNKS)
        def _():
            base = c * _N_ROWS
            pltpu.sync_copy(nb_hbm.at[pl.ds(c * _N_K, _N_K)], ni)

            w = {}
            for j in range(_N_K):
                if j >= 2:
                    w.pop(j - 2).wait()
                pltpu.async_copy(xu_spm.at[ni.at[j]], gn.at[j & 1],
                                 gsem).wait()
                w[j] = pltpu.async_copy(
                    gn.at[j & 1], on_hbm.at[pl.ds(base + j * _N_W, _N_W)],
                    wsem)
            for j in (_N_K - 2, _N_K - 1):
                w.pop(j).wait()


@functools.partial(
    pl.kernel,
    out_type=(
        jax.ShapeDtypeStruct((2, N_V, D), jnp.float32),
        jax.ShapeDtypeStruct((2, N_G, D), jnp.float32),
    ),
    mesh=_MESH,
    compiler_params=pltpu.CompilerParams(use_tc_tiling_on_sc=False),
    scratch_types=[
        pltpu.VMEM_SHARED((N_V, D), jnp.float32),
        pltpu.VMEM_SHARED((N_G, D), jnp.float32),
        pltpu.VMEM((_E_K, _E_W), jnp.int32),
        pltpu.VMEM((_E_K, _E_W), jnp.int32),
        pltpu.VMEM((4, _E_W, D), jnp.float32),
        pltpu.SemaphoreType.DMA,
        pltpu.SemaphoreType.DMA,
    ],
)
def _sc_scatter_edges(e_hbm, dst_hbm, bb_hbm, z_hbm, ov_hbm, ou_hbm,
                      accv, accu, di, bi, ge, lsem, ssem):
    cid = lax.axis_index("c")
    sid = lax.axis_index("s")
    wid = _worker_id()

    @pl.when(sid < _VSUB)
    def _():
        pltpu.sync_copy(z_hbm.at[pl.ds(sid * _VROWS, _VROWS)],
                        accv.at[pl.ds(sid * _VROWS, _VROWS)])

    pltpu.sync_copy(z_hbm.at[pl.ds(sid * _GROWS, _GROWS)],
                    accu.at[pl.ds(sid * _GROWS, _GROWS)])
    plsc.subcore_barrier()

    @pl.loop(0, _E_TRIPS)
    def _(t):
        c = wid + t * _NW

        @pl.when(c < _E_CHUNKS)
        def _():
            base = c * _E_ROWS
            pltpu.sync_copy(dst_hbm.at[pl.ds(c * _E_K, _E_K)], di)
            pltpu.sync_copy(bb_hbm.at[pl.ds(c * _E_K, _E_K)], bi)

            def load(j):
                return pltpu.async_copy(
                    e_hbm.at[pl.ds(base + j * _E_W, _E_W)], ge.at[j & 3],
                    lsem)

            ld = {0: load(0), 1: load(1)}
            for j in range(_E_K):
                b = j & 3
                ld.pop(j).wait()
                if j + 2 < _E_K:
                    ld[j + 2] = load(j + 2)
                pltpu.async_copy(ge.at[b], accv.at[di.at[j]], ssem,
                                 add=True).wait()
                pltpu.async_copy(ge.at[b], accu.at[bi.at[j]], ssem,
                                 add=True).wait()

    plsc.subcore_barrier()

    @pl.when(sid < _VSUB)
    def _():
        pltpu.sync_copy(accv.at[pl.ds(sid * _VROWS, _VROWS)],
                        ov_hbm.at[cid, pl.ds(sid * _VROWS, _VROWS)])

    pltpu.sync_copy(accu.at[pl.ds(sid * _GROWS, _GROWS)],
                    ou_hbm.at[cid, pl.ds(sid * _GROWS, _GROWS)])


@functools.partial(
    pl.kernel,
    out_type=jax.ShapeDtypeStruct((2, N_G, D), jnp.float32),
    mesh=_MESH,
    compiler_params=pltpu.CompilerParams(use_tc_tiling_on_sc=False),
    scratch_types=[
        pltpu.VMEM_SHARED((N_G, D), jnp.float32),
        pltpu.VMEM((_N_K, _N_W), jnp.int32),
        pltpu.VMEM((4, _N_W, D), jnp.float32),
        pltpu.SemaphoreType.DMA,
        pltpu.SemaphoreType.DMA,
    ],
)
def _sc_scatter_nodes(v_hbm, nb_hbm, z_hbm, ou_hbm, accu, ni, gv, lsem, ssem):
    cid = lax.axis_index("c")
    sid = lax.axis_index("s")
    wid = _worker_id()
    pltpu.sync_copy(z_hbm.at[pl.ds(sid * _GROWS, _GROWS)],
                    accu.at[pl.ds(sid * _GROWS, _GROWS)])
    plsc.subcore_barrier()

    @pl.loop(0, _N_TRIPS)
    def _(t):
        c = wid + t * _NW

        @pl.when(c < _N_CHUNKS)
        def _():
            base = c * _N_ROWS
            pltpu.sync_copy(nb_hbm.at[pl.ds(c * _N_K, _N_K)], ni)

            def load(j):
                return pltpu.async_copy(
                    v_hbm.at[pl.ds(base + j * _N_W, _N_W)], gv.at[j & 3],
                    lsem)

            ld = {0: load(0), 1: load(1)}
            for j in range(_N_K):
                ld.pop(j).wait()
                if j + 2 < _N_K:
                    ld[j + 2] = load(j + 2)
                pltpu.async_copy(gv.at[j & 3], accu.at[ni.at[j]], ssem,
                                 add=True).wait()

    plsc.subcore_barrier()
    pltpu.sync_copy(accu.at[pl.ds(sid * _GROWS, _GROWS)],
                    ou_hbm.at[cid, pl.ds(sid * _GROWS, _GROWS)])


@functools.partial(
    pl.kernel,
    out_type=(
        jax.ShapeDtypeStruct((2, N_V), jnp.float32),
        jax.ShapeDtypeStruct((2, N_G), jnp.float32),
        jax.ShapeDtypeStruct((2, N_G), jnp.float32),
    ),
    mesh=_MESH,
    compiler_params=pltpu.CompilerParams(use_tc_tiling_on_sc=False),
    scratch_types=[
        pltpu.VMEM_SHARED((N_V,), jnp.float32),
        pltpu.VMEM_SHARED((N_G,), jnp.float32),
        pltpu.VMEM_SHARED((N_G,), jnp.float32),
        pltpu.VMEM((_E_K, _E_W), jnp.int32),
        pltpu.VMEM((_E_K, _E_W), jnp.int32),
        pltpu.VMEM((_N_K, _N_W), jnp.int32),
        pltpu.VMEM((128,), jnp.float32),
        pltpu.SemaphoreType.DMA,
    ],
)
def _sc_counts(dst_hbm, bb_hbm, nb_hbm, z_hbm, od_hbm, oe_hbm, ov_hbm,
               accd, acce, accv, di, bi, ni, ones, ssem):
    cid = lax.axis_index("c")
    sid = lax.axis_index("s")
    wid = _worker_id()
    for j in range(8):
        ones[pl.ds(j * 16, 16)] = jnp.ones((16,), jnp.float32)

    @pl.when(sid < _VSUB)
    def _():
        pltpu.sync_copy(z_hbm.at[pl.ds(sid * _VROWS, _VROWS)],
                        accd.at[pl.ds(sid * _VROWS, _VROWS)])

    pltpu.sync_copy(z_hbm.at[pl.ds(sid * _GROWS, _GROWS)],
                    acce.at[pl.ds(sid * _GROWS, _GROWS)])
    pltpu.sync_copy(z_hbm.at[pl.ds(sid * _GROWS, _GROWS)],
                    accv.at[pl.ds(sid * _GROWS, _GROWS)])
    plsc.subcore_barrier()

    @pl.loop(0, _E_TRIPS)
    def _(t):
        c = wid + t * _NW

        @pl.when(c < _E_CHUNKS)
        def _():
            pltpu.sync_copy(dst_hbm.at[pl.ds(c * _E_K, _E_K)], di)
            pltpu.sync_copy(bb_hbm.at[pl.ds(c * _E_K, _E_K)], bi)
            for j in range(_E_K):
                pltpu.async_copy(ones.at[pl.ds(0, _E_W)], accd.at[di.at[j]],
                                 ssem, add=True).wait()
                pltpu.async_copy(ones.at[pl.ds(0, _E_W)], acce.at[bi.at[j]],
                                 ssem, add=True).wait()

    @pl.loop(0, _N_TRIPS)
    def _(t):
        c = wid + t * _NW

        @pl.when(c < _N_CHUNKS)
        def _():
            pltpu.sync_copy(nb_hbm.at[pl.ds(c * _N_K, _N_K)], ni)
            for j in range(_N_K):
                pltpu.async_copy(ones.at[pl.ds(0, _N_W)], accv.at[ni.at[j]],
                                 ssem, add=True).wait()

    plsc.subcore_barrier()

    @pl.when(sid < _VSUB)
    def _():
        pltpu.sync_copy(accd.at[pl.ds(sid * _VROWS, _VROWS)],
                        od_hbm.at[cid, pl.ds(sid * _VROWS, _VROWS)])

    pltpu.sync_copy(acce.at[pl.ds(sid * _GROWS, _GROWS)],
                    oe_hbm.at[cid, pl.ds(sid * _GROWS, _GROWS)])
    pltpu.sync_copy(accv.at[pl.ds(sid * _GROWS, _GROWS)],
                    ov_hbm.at[cid, pl.ds(sid * _GROWS, _GROWS)])


_R = 2000


def _full(a):
    if a.ndim == 3:
        return pl.BlockSpec(a.shape, lambda i: (0, 0, 0))
    if a.ndim == 2:
        return pl.BlockSpec(a.shape, lambda i: (0, 0))
    return pl.BlockSpec(a.shape, lambda i: (0,))


def _sp(x):
    return jax.nn.softplus(x)


def _pre_v_kernel(x_ref, w1, b1, w2, b2, o_ref):
    h = _sp(x_ref[...] @ w1[...] + b1[...])
    o_ref[...] = _sp(h @ w2[...] + b2[...])


def _pre_v(x, p):
    w1, b1 = p[0]["W"], p[0]["b"][None]
    w2, b2 = p[1]["W"], p[1]["b"][None]
    din = x.shape[1]
    return pl.pallas_call(
        _pre_v_kernel,
        grid=(N_V // _R,),
        in_specs=[pl.BlockSpec((_R, din), lambda i: (i, 0))]
        + [_full(w) for w in (w1, b1, w2, b2)],
        out_specs=pl.BlockSpec((_R, D), lambda i: (i, 0)),
        out_shape=jax.ShapeDtypeStruct((N_V, D), jnp.float32),
    )(x, w1, b1, w2, b2)


def _pre_u_kernel(u_ref, w1, b1, w2, b2, o_ref):
    h = _sp(u_ref[...] @ w1[...] + b1[...])
    o_ref[...] = _sp(h @ w2[...] + b2[...])


def _pre_u(u, p):
    w1, b1 = p[0]["W"], p[0]["b"][None]
    w2, b2 = p[1]["W"], p[1]["b"][None]
    return pl.pallas_call(
        _pre_u_kernel,
        out_shape=jax.ShapeDtypeStruct((N_G, D), jnp.float32),
    )(u, w1, b1, w2, b2)


def _edge_body(ea, xvs, xvd, xub, w1, b1, w2, b2, wa, wb, wc, wd, bp, v2, c2,
               v3, c3):
    xe = _sp(ea[...] @ w1[...] + b1[...])
    xe = _sp(xe @ w2[...] + b2[...])
    h = _sp(xvs[...] @ wa[...] + xvd[...] @ wb[...] + xe @ wc[...]
            + xub[...] @ wd[...] + bp[...])
    h = _sp(h @ v2[...] + c2[...])
    return _sp(h @ v3[...] + c3[...])


def _edge_kernel_skip(ea, xvs, xvd, xub, w1, b1, w2, b2, wa, wb, wc, wd, bp,
                      v2, c2, v3, c3, o_pre, o_post):
    e_pre = _edge_body(ea, xvs, xvd, xub, w1, b1, w2, b2, wa, wb, wc, wd, bp,
                       v2, c2, v3, c3)
    o_pre[...] = e_pre
    o_post[...] = e_pre + ea[...]


def _edge_kernel_noskip(ea, xvs, xvd, xub, w1, b1, w2, b2, wa, wb, wc, wd, bp,
                        v2, c2, v3, c3, o_pre):
    o_pre[...] = _edge_body(ea, xvs, xvd, xub, w1, b1, w2, b2, wa, wb, wc, wd,
                            bp, v2, c2, v3, c3)


def _edge_mlp(ea, xvs, xvd, xub, p, skip_out):
    w1, b1 = p["pre_e"][0]["W"], p["pre_e"][0]["b"][None]
    w2, b2 = p["pre_e"][1]["W"], p["pre_e"][1]["b"][None]
    W = p["phi_e"][0]["W"]
    wa, wb, wc, wd = W[0:D], W[D:2 * D], W[2 * D:3 * D], W[3 * D:4 * D]
    bp = p["phi_e"][0]["b"][None]
    v2, c2 = p["phi_e"][1]["W"], p["phi_e"][1]["b"][None]
    v3, c3 = p["phi_e"][2]["W"], p["phi_e"][2]["b"][None]
    din = ea.shape[1]
    n_out = 2 if skip_out else 1
    ws = (w1, b1, w2, b2, wa, wb, wc, wd, bp, v2, c2, v3, c3)
    outs = pl.pallas_call(
        _edge_kernel_skip if skip_out else _edge_kernel_noskip,
        grid=(N_E // _R,),
        in_specs=[pl.BlockSpec((_R, din), lambda i: (i, 0))]
        + [pl.BlockSpec((_R, D), lambda i: (i, 0))] * 3
        + [_full(w) for w in ws],
        out_specs=[pl.BlockSpec((_R, D), lambda i: (i, 0))] * n_out,
        out_shape=[jax.ShapeDtypeStruct((N_E, D), jnp.float32)] * n_out,
    )(ea, xvs, xvd, xub, *ws)
    return outs if skip_out else (outs[0], outs[0])


def _phi_v_kernel_skip(xv, evp, degp, xun, xin, wa, wb, wc, bp, v2, c2, v3, c3,
                       o_pre, o_post):
    r = 1.0 / jnp.maximum(degp[0, :, :] + degp[1, :, :], 1.0)
    etov = (evp[0] + evp[1]) * r
    h = _sp(xv[...] @ wa[...] + etov @ wb[...] + xun[...] @ wc[...] + bp[...])
    h = _sp(h @ v2[...] + c2[...])
    v_pre = _sp(h @ v3[...] + c3[...])
    o_pre[...] = v_pre
    o_post[...] = v_pre + xin[...]


def _phi_v_kernel_noskip(xv, evp, degp, xun, wa, wb, wc, bp, v2, c2, v3, c3,
                         o_pre):
    r = 1.0 / jnp.maximum(degp[0, :, :] + degp[1, :, :], 1.0)
    etov = (evp[0] + evp[1]) * r
    h = _sp(xv[...] @ wa[...] + etov @ wb[...] + xun[...] @ wc[...] + bp[...])
    h = _sp(h @ v2[...] + c2[...])
    o_pre[...] = _sp(h @ v3[...] + c3[...])


def _phi_v(xv, evp, degp, xun, xin, p, skip_out):
    W = p["phi_v"][0]["W"]
    wa, wb, wc = W[0:D], W[D:2 * D], W[2 * D:3 * D]
    bp = p["phi_v"][0]["b"][None]
    v2, c2 = p["phi_v"][1]["W"], p["phi_v"][1]["b"][None]
    v3, c3 = p["phi_v"][2]["W"], p["phi_v"][2]["b"][None]
    degp3 = degp[:, :, None]
    ws = (wa, wb, wc, bp, v2, c2, v3, c3)
    n_out = 2 if skip_out else 1
    row = pl.BlockSpec((_R, D), lambda i: (i, 0))
    ins = [xv, evp, degp3, xun] + ([xin] if skip_out else [])
    in_specs = [row,
                pl.BlockSpec((2, _R, D), lambda i: (0, i, 0)),
                pl.BlockSpec((2, _R, 1), lambda i: (0, i, 0)),
                row] + ([row] if skip_out else [])
    outs = pl.pallas_call(
        _phi_v_kernel_skip if skip_out else _phi_v_kernel_noskip,
        grid=(N_V // _R,),
        in_specs=in_specs + [_full(w) for w in ws],
        out_specs=[row] * n_out,
        out_shape=[jax.ShapeDtypeStruct((N_V, D), jnp.float32)] * n_out,
    )(*ins, *ws)
    return outs if skip_out else (outs[0], outs[0])


def _phi_u_kernel(uep, uvp, cep, cvp, xu, uin, wa, wb, wc, bp, v2, c2, v3, c3,
                  o_ref):
    ue = (uep[0] + uep[1]) / jnp.maximum(cep[0] + cep[1], 1.0)
    uv = (uvp[0] + uvp[1]) / jnp.maximum(cvp[0] + cvp[1], 1.0)
    h = _sp(ue @ wa[...] + uv @ wb[...] + xu[...] @ wc[...] + bp[...])
    h = _sp(h @ v2[...] + c2[...])
    u_pre = _sp(h @ v3[...] + c3[...])
    o_ref[...] = u_pre + uin[...]


def _phi_u(uep, uvp, cep, cvp, xu, uin, p):
    W = p["phi_u"][0]["W"]
    wa, wb, wc = W[0:D], W[D:2 * D], W[2 * D:3 * D]
    bp = p["phi_u"][0]["b"][None]
    v2, c2 = p["phi_u"][1]["W"], p["phi_u"][1]["b"][None]
    v3, c3 = p["phi_u"][2]["W"], p["phi_u"][2]["b"][None]
    return pl.pallas_call(
        _phi_u_kernel,
        out_shape=jax.ShapeDtypeStruct((N_G, D), jnp.float32),
    )(uep, uvp, cep[:, :, None], cvp[:, :, None], xu, uin,
      wa, wb, wc, bp, v2, c2, v3, c3)


def _head_kernel(uv1, uv2, uv3, ue1, ue2, ue3, cep, cvp, uu3,
                 w0, b0, w1, b1, w2, b2, o_ref):
    mv_sum = uv1[0] + uv1[1] + uv2[0] + uv2[1] + uv3[0] + uv3[1]
    me_sum = ue1[0] + ue1[1] + ue2[0] + ue2[1] + ue3[0] + ue3[1]
    mv = mv_sum / jnp.maximum(cvp[0] + cvp[1], 1.0)
    me = me_sum / jnp.maximum(cep[0] + cep[1], 1.0)
    z = jnp.zeros_like(mv)
    tmp = jnp.concatenate([z, mv, z, me, uu3[...]], axis=1)
    h = _sp(tmp @ w0[...] + b0[...])
    h = _sp(h @ w1[...] + b1[...])
    o_ref[...] = h @ w2[...] + b2[...]


def _module(x, edge_attr, state_feat, uin, idx2d, counts, p, skip):
    src2d, dst2d, bb2d, nb2d, zeros = idx2d
    degp, cep, cvp = counts
    xv = _pre_v(x, p["pre_v"])
    xu = _pre_u(state_feat, p["pre_u"])
    xvs, xvd, xub, xun = _sc_gather(xv, xu, src2d, dst2d, bb2d, nb2d)
    e_pre, e_post = _edge_mlp(edge_attr, xvs, xvd, xub, p, skip)
    evp, uep = _sc_scatter_edges(e_pre, dst2d, bb2d, zeros)
    v_pre, x_post = _phi_v(xv, evp, degp, xun, x, p, skip)
    uvp = _sc_scatter_nodes(v_pre, nb2d, zeros)
    uu = _phi_u(uep, uvp, cep, cvp, xu, uin, p)
    return x_post, e_post, uu, uvp, uep


def kernel(x, edge_index, edge_attr, state, batch, bond_batch, params):
    src2d = edge_index[0].reshape(_E_CHUNKS * _E_K, _E_W)
    dst2d = edge_index[1].reshape(_E_CHUNKS * _E_K, _E_W)
    bb2d = bond_batch.reshape(_E_CHUNKS * _E_K, _E_W)
    nb2d = batch.reshape(_N_CHUNKS * _N_K, _N_W)
    zeros = jnp.zeros((N_V, D), jnp.float32)
    zeros1 = jnp.zeros((N_V,), jnp.float32)
    zg = jnp.zeros((N_G, D), jnp.float32)

    degp, cep, cvp = _sc_counts(dst2d, bb2d, nb2d, zeros1)
    counts = (degp, cep, cvp)
    idx2d = (src2d, dst2d, bb2d, nb2d, zeros)

    x1, ee1, uu1, uvp1, uep1 = _module(x, edge_attr, state, zg, idx2d,
                                       counts, params["m1"], False)
    x2, ee2, uu2, uvp2, uep2 = _module(x1, ee1, uu1, uu1, idx2d,
                                       counts, params["m2"], True)
    _, _, uu3, uvp3, uep3 = _module(x2, ee2, uu2, uu2, idx2d,
                                    counts, params["m3"], False)

    hl = params["hiddens"]
    out = pl.pallas_call(
        _head_kernel,
        out_shape=jax.ShapeDtypeStruct((N_G, 1), jnp.float32),
    )(uvp1, uvp2, uvp3, uep1, uep2, uep3, cep[:, :, None], cvp[:, :, None],
      uu3, hl[0]["W"], hl[0]["b"][None], hl[1]["W"], hl[1]["b"][None],
      hl[2]["W"], hl[2]["b"][None])
    return out


# speedup vs baseline: 1.1020x; 1.0019x over previous
import functools

import jax
import jax.numpy as jnp
from jax import lax
from jax.experimental import pallas as pl
from jax.experimental.pallas import tpu as pltpu
from jax.experimental.pallas import tpu_sc as plsc

N_V = 50000
N_E = 800000
N_G = 512
D = 32

_E_W = 100
_E_K = 8
_E_ROWS = _E_W * _E_K
_E_CHUNKS = N_E // _E_ROWS
_N_W = 25
_N_K = 8
_N_ROWS = _N_W * _N_K
_N_CHUNKS = N_V // _N_ROWS

_NW = 32
_E_TRIPS = (_E_CHUNKS + _NW - 1) // _NW
_N_TRIPS = (_N_CHUNKS + _NW - 1) // _NW

_MESH = plsc.VectorSubcoreMesh(core_axis_name="c", subcore_axis_name="s")
_NSUB = 16
_VSUB = 10
_VROWS = N_V // _VSUB
_GROWS = N_G // _NSUB


def _worker_id():
    return lax.axis_index("s") * 2 + lax.axis_index("c")


@functools.partial(
    pl.kernel,
    out_type=(
        jax.ShapeDtypeStruct((N_E, D), jnp.float32),
        jax.ShapeDtypeStruct((N_E, D), jnp.float32),
        jax.ShapeDtypeStruct((N_E, D), jnp.float32),
        jax.ShapeDtypeStruct((N_V, D), jnp.float32),
    ),
    mesh=_MESH,
    compiler_params=pltpu.CompilerParams(use_tc_tiling_on_sc=False),
    scratch_types=[
        pltpu.VMEM_SHARED((N_V, D), jnp.float32),
        pltpu.VMEM_SHARED((N_G, D), jnp.float32),
        pltpu.VMEM((_E_K, _E_W), jnp.int32),
        pltpu.VMEM((_E_K, _E_W), jnp.int32),
        pltpu.VMEM((_E_K, _E_W), jnp.int32),
        pltpu.VMEM((2, _E_W, D), jnp.float32),
        pltpu.VMEM((2, _E_W, D), jnp.float32),
        pltpu.VMEM((2, _E_W, D), jnp.float32),
        pltpu.VMEM((_N_K, _N_W), jnp.int32),
        pltpu.VMEM((2, _N_W, D), jnp.float32),
        pltpu.SemaphoreType.DMA,
        pltpu.SemaphoreType.DMA,
    ],
)
def _sc_gather(xv_hbm, xu_hbm, src_hbm, dst_hbm, bb_hbm, nb_hbm,
               os_hbm, od_hbm, ob_hbm, on_hbm,
               xv_spm, xu_spm, si, di, bi, gs, gd, gb, ni, gn, gsem, wsem):
    sid = lax.axis_index("s")
    wid = _worker_id()

    @pl.when(sid < _VSUB)
    def _():
        pltpu.sync_copy(xv_hbm.at[pl.ds(sid * _VROWS, _VROWS)],
                        xv_spm.at[pl.ds(sid * _VROWS, _VROWS)])

    pltpu.sync_copy(xu_hbm.at[pl.ds(sid * _GROWS, _GROWS)],
                    xu_spm.at[pl.ds(sid * _GROWS, _GROWS)])
    plsc.subcore_barrier()

    @pl.loop(0, _E_TRIPS)
    def _(t):
        c = wid + t * _NW

        @pl.when(c < _E_CHUNKS)
        def _():
            base = c * _E_ROWS
            pltpu.sync_copy(src_hbm.at[pl.ds(c * _E_K, _E_K)], si)
            pltpu.sync_copy(dst_hbm.at[pl.ds(c * _E_K, _E_K)], di)
            pltpu.sync_copy(bb_hbm.at[pl.ds(c * _E_K, _E_K)], bi)

            w = {}
            for j in range(_E_K):
                b = j & 1
                ww = pl.ds(base + j * _E_W, _E_W)
                if j >= 2:
                    for h in w.pop(j - 2):
                        h.wait()
                h1 = pltpu.async_copy(xv_spm.at[si.at[j]], gs.at[b], gsem)
                h2 = pltpu.async_copy(xv_spm.at[di.at[j]], gd.at[b], gsem)
                h3 = pltpu.async_copy(xu_spm.at[bi.at[j]], gb.at[b], gsem)
                h1.wait()
                h2.wait()
                h3.wait()
                w[j] = (
                    pltpu.async_copy(gs.at[b], os_hbm.at[ww], wsem),
                    pltpu.async_copy(gd.at[b], od_hbm.at[ww], wsem),
                    pltpu.async_copy(gb.at[b], ob_hbm.at[ww], wsem),
                )
            for j in (_E_K - 2, _E_K - 1):
                for h in w.pop(j):
                    h.wait()

    @pl.loop(0, _N_TRIPS)
    def _(t):
        c = wid + t * _NW

        @pl.when(c < _N_CHUNKS)
        def _():
            base = c * _N_ROWS
            pltpu.sync_copy(nb_hbm.at[pl.ds(c * _N_K, _N_K)], ni)

            w = {}
            for j in range(_N_K):
                if j >= 2:
                    w.pop(j - 2).wait()
                pltpu.async_copy(xu_spm.at[ni.at[j]], gn.at[j & 1],
                                 gsem).wait()
                w[j] = pltpu.async_copy(
                    gn.at[j & 1], on_hbm.at[pl.ds(base + j * _N_W, _N_W)],
                    wsem)
            for j in (_N_K - 2, _N_K - 1):
                w.pop(j).wait()


@functools.partial(
    pl.kernel,
    out_type=(
        jax.ShapeDtypeStruct((2, N_V, D), jnp.float32),
        jax.ShapeDtypeStruct((2, N_G, D), jnp.float32),
    ),
    mesh=_MESH,
    compiler_params=pltpu.CompilerParams(use_tc_tiling_on_sc=False),
    scratch_types=[
        pltpu.VMEM_SHARED((N_V, D), jnp.float32),
        pltpu.VMEM_SHARED((N_G, D), jnp.float32),
        pltpu.VMEM((_E_K, _E_W), jnp.int32),
        pltpu.VMEM((_E_K, _E_W), jnp.int32),
        pltpu.VMEM((4, _E_W, D), jnp.float32),
        pltpu.SemaphoreType.DMA,
        pltpu.SemaphoreType.DMA,
    ],
)
def _sc_scatter_edges(e_hbm, dst_hbm, bb_hbm, z_hbm, ov_hbm, ou_hbm,
                      accv, accu, di, bi, ge, lsem, ssem):
    cid = lax.axis_index("c")
    sid = lax.axis_index("s")
    wid = _worker_id()

    @pl.when(sid < _VSUB)
    def _():
        pltpu.sync_copy(z_hbm.at[pl.ds(sid * _VROWS, _VROWS)],
                        accv.at[pl.ds(sid * _VROWS, _VROWS)])

    pltpu.sync_copy(z_hbm.at[pl.ds(sid * _GROWS, _GROWS)],
                    accu.at[pl.ds(sid * _GROWS, _GROWS)])
    plsc.subcore_barrier()

    @pl.loop(0, _E_TRIPS)
    def _(t):
        c = wid + t * _NW

        @pl.when(c < _E_CHUNKS)
        def _():
            base = c * _E_ROWS
            pltpu.sync_copy(dst_hbm.at[pl.ds(c * _E_K, _E_K)], di)
            pltpu.sync_copy(bb_hbm.at[pl.ds(c * _E_K, _E_K)], bi)

            def load(j):
                return pltpu.async_copy(
                    e_hbm.at[pl.ds(base + j * _E_W, _E_W)], ge.at[j & 3],
                    lsem)

            ld = {0: load(0), 1: load(1)}
            for j in range(_E_K):
                b = j & 3
                ld.pop(j).wait()
                if j + 2 < _E_K:
                    ld[j + 2] = load(j + 2)
                h1 = pltpu.async_copy(ge.at[b], accv.at[di.at[j]], ssem,
                                      add=True)
                h2 = pltpu.async_copy(ge.at[b], accu.at[bi.at[j]], ssem,
                                      add=True)
                h1.wait()
                h2.wait()

    plsc.subcore_barrier()

    @pl.when(sid < _VSUB)
    def _():
        pltpu.sync_copy(accv.at[pl.ds(sid * _VROWS, _VROWS)],
                        ov_hbm.at[cid, pl.ds(sid * _VROWS, _VROWS)])

    pltpu.sync_copy(accu.at[pl.ds(sid * _GROWS, _GROWS)],
                    ou_hbm.at[cid, pl.ds(sid * _GROWS, _GROWS)])


@functools.partial(
    pl.kernel,
    out_type=jax.ShapeDtypeStruct((2, N_G, D), jnp.float32),
    mesh=_MESH,
    compiler_params=pltpu.CompilerParams(use_tc_tiling_on_sc=False),
    scratch_types=[
        pltpu.VMEM_SHARED((N_G, D), jnp.float32),
        pltpu.VMEM((_N_K, _N_W), jnp.int32),
        pltpu.VMEM((4, _N_W, D), jnp.float32),
        pltpu.SemaphoreType.DMA,
        pltpu.SemaphoreType.DMA,
    ],
)
def _sc_scatter_nodes(v_hbm, nb_hbm, z_hbm, ou_hbm, accu, ni, gv, lsem, ssem):
    cid = lax.axis_index("c")
    sid = lax.axis_index("s")
    wid = _worker_id()
    pltpu.sync_copy(z_hbm.at[pl.ds(sid * _GROWS, _GROWS)],
                    accu.at[pl.ds(sid * _GROWS, _GROWS)])
    plsc.subcore_barrier()

    @pl.loop(0, _N_TRIPS)
    def _(t):
        c = wid + t * _NW

        @pl.when(c < _N_CHUNKS)
        def _():
            base = c * _N_ROWS
            pltpu.sync_copy(nb_hbm.at[pl.ds(c * _N_K, _N_K)], ni)

            def load(j):
                return pltpu.async_copy(
                    v_hbm.at[pl.ds(base + j * _N_W, _N_W)], gv.at[j & 3],
                    lsem)

            ld = {0: load(0), 1: load(1)}
            for j in range(_N_K):
                ld.pop(j).wait()
                if j + 2 < _N_K:
                    ld[j + 2] = load(j + 2)
                pltpu.async_copy(gv.at[j & 3], accu.at[ni.at[j]], ssem,
                                 add=True).wait()

    plsc.subcore_barrier()
    pltpu.sync_copy(accu.at[pl.ds(sid * _GROWS, _GROWS)],
                    ou_hbm.at[cid, pl.ds(sid * _GROWS, _GROWS)])


@functools.partial(
    pl.kernel,
    out_type=(
        jax.ShapeDtypeStruct((2, N_V), jnp.float32),
        jax.ShapeDtypeStruct((2, N_G), jnp.float32),
        jax.ShapeDtypeStruct((2, N_G), jnp.float32),
    ),
    mesh=_MESH,
    compiler_params=pltpu.CompilerParams(use_tc_tiling_on_sc=False),
    scratch_types=[
        pltpu.VMEM_SHARED((N_V,), jnp.float32),
        pltpu.VMEM_SHARED((N_G,), jnp.float32),
        pltpu.VMEM_SHARED((N_G,), jnp.float32),
        pltpu.VMEM((_E_K, _E_W), jnp.int32),
        pltpu.VMEM((_E_K, _E_W), jnp.int32),
        pltpu.VMEM((_N_K, _N_W), jnp.int32),
        pltpu.VMEM((128,), jnp.float32),
        pltpu.SemaphoreType.DMA,
    ],
)
def _sc_counts(dst_hbm, bb_hbm, nb_hbm, z_hbm, od_hbm, oe_hbm, ov_hbm,
               accd, acce, accv, di, bi, ni, ones, ssem):
    cid = lax.axis_index("c")
    sid = lax.axis_index("s")
    wid = _worker_id()
    for j in range(8):
        ones[pl.ds(j * 16, 16)] = jnp.ones((16,), jnp.float32)

    @pl.when(sid < _VSUB)
    def _():
        pltpu.sync_copy(z_hbm.at[pl.ds(sid * _VROWS, _VROWS)],
                        accd.at[pl.ds(sid * _VROWS, _VROWS)])

    pltpu.sync_copy(z_hbm.at[pl.ds(sid * _GROWS, _GROWS)],
                    acce.at[pl.ds(sid * _GROWS, _GROWS)])
    pltpu.sync_copy(z_hbm.at[pl.ds(sid * _GROWS, _GROWS)],
                    accv.at[pl.ds(sid * _GROWS, _GROWS)])
    plsc.subcore_barrier()

    @pl.loop(0, _E_TRIPS)
    def _(t):
        c = wid + t * _NW

        @pl.when(c < _E_CHUNKS)
        def _():
            pltpu.sync_copy(dst_hbm.at[pl.ds(c * _E_K, _E_K)], di)
            pltpu.sync_copy(bb_hbm.at[pl.ds(c * _E_K, _E_K)], bi)
            for j in range(_E_K):
                h1 = pltpu.async_copy(ones.at[pl.ds(0, _E_W)],
                                      accd.at[di.at[j]], ssem, add=True)
                h2 = pltpu.async_copy(ones.at[pl.ds(0, _E_W)],
                                      acce.at[bi.at[j]], ssem, add=True)
                h1.wait()
                h2.wait()

    @pl.loop(0, _N_TRIPS)
    def _(t):
        c = wid + t * _NW

        @pl.when(c < _N_CHUNKS)
        def _():
            pltpu.sync_copy(nb_hbm.at[pl.ds(c * _N_K, _N_K)], ni)
            for j in range(_N_K):
                pltpu.async_copy(ones.at[pl.ds(0, _N_W)], accv.at[ni.at[j]],
                                 ssem, add=True).wait()

    plsc.subcore_barrier()

    @pl.when(sid < _VSUB)
    def _():
        pltpu.sync_copy(accd.at[pl.ds(sid * _VROWS, _VROWS)],
                        od_hbm.at[cid, pl.ds(sid * _VROWS, _VROWS)])

    pltpu.sync_copy(acce.at[pl.ds(sid * _GROWS, _GROWS)],
                    oe_hbm.at[cid, pl.ds(sid * _GROWS, _GROWS)])
    pltpu.sync_copy(accv.at[pl.ds(sid * _GROWS, _GROWS)],
                    ov_hbm.at[cid, pl.ds(sid * _GROWS, _GROWS)])


_R = 2000


def _full(a):
    if a.ndim == 3:
        return pl.BlockSpec(a.shape, lambda i: (0, 0, 0))
    if a.ndim == 2:
        return pl.BlockSpec(a.shape, lambda i: (0, 0))
    return pl.BlockSpec(a.shape, lambda i: (0,))


def _sp(x):
    return jax.nn.softplus(x)


def _pre_v_kernel(x_ref, w1, b1, w2, b2, o_ref):
    h = _sp(x_ref[...] @ w1[...] + b1[...])
    o_ref[...] = _sp(h @ w2[...] + b2[...])


def _pre_v(x, p):
    w1, b1 = p[0]["W"], p[0]["b"][None]
    w2, b2 = p[1]["W"], p[1]["b"][None]
    din = x.shape[1]
    return pl.pallas_call(
        _pre_v_kernel,
        grid=(N_V // _R,),
        in_specs=[pl.BlockSpec((_R, din), lambda i: (i, 0))]
        + [_full(w) for w in (w1, b1, w2, b2)],
        out_specs=pl.BlockSpec((_R, D), lambda i: (i, 0)),
        out_shape=jax.ShapeDtypeStruct((N_V, D), jnp.float32),
    )(x, w1, b1, w2, b2)


def _pre_u_kernel(u_ref, w1, b1, w2, b2, o_ref):
    h = _sp(u_ref[...] @ w1[...] + b1[...])
    o_ref[...] = _sp(h @ w2[...] + b2[...])


def _pre_u(u, p):
    w1, b1 = p[0]["W"], p[0]["b"][None]
    w2, b2 = p[1]["W"], p[1]["b"][None]
    return pl.pallas_call(
        _pre_u_kernel,
        out_shape=jax.ShapeDtypeStruct((N_G, D), jnp.float32),
    )(u, w1, b1, w2, b2)


def _edge_body(ea, xvs, xvd, xub, w1, b1, w2, b2, wa, wb, wc, wd, bp, v2, c2,
               v3, c3):
    xe = _sp(ea[...] @ w1[...] + b1[...])
    xe = _sp(xe @ w2[...] + b2[...])
    h = _sp(xvs[...] @ wa[...] + xvd[...] @ wb[...] + xe @ wc[...]
            + xub[...] @ wd[...] + bp[...])
    h = _sp(h @ v2[...] + c2[...])
    return _sp(h @ v3[...] + c3[...])


def _edge_kernel_skip(ea, xvs, xvd, xub, w1, b1, w2, b2, wa, wb, wc, wd, bp,
                      v2, c2, v3, c3, o_pre, o_post):
    e_pre = _edge_body(ea, xvs, xvd, xub, w1, b1, w2, b2, wa, wb, wc, wd, bp,
                       v2, c2, v3, c3)
    o_pre[...] = e_pre
    o_post[...] = e_pre + ea[...]


def _edge_kernel_noskip(ea, xvs, xvd, xub, w1, b1, w2, b2, wa, wb, wc, wd, bp,
                        v2, c2, v3, c3, o_pre):
    o_pre[...] = _edge_body(ea, xvs, xvd, xub, w1, b1, w2, b2, wa, wb, wc, wd,
                            bp, v2, c2, v3, c3)


def _edge_mlp(ea, xvs, xvd, xub, p, skip_out):
    w1, b1 = p["pre_e"][0]["W"], p["pre_e"][0]["b"][None]
    w2, b2 = p["pre_e"][1]["W"], p["pre_e"][1]["b"][None]
    W = p["phi_e"][0]["W"]
    wa, wb, wc, wd = W[0:D], W[D:2 * D], W[2 * D:3 * D], W[3 * D:4 * D]
    bp = p["phi_e"][0]["b"][None]
    v2, c2 = p["phi_e"][1]["W"], p["phi_e"][1]["b"][None]
    v3, c3 = p["phi_e"][2]["W"], p["phi_e"][2]["b"][None]
    din = ea.shape[1]
    n_out = 2 if skip_out else 1
    ws = (w1, b1, w2, b2, wa, wb, wc, wd, bp, v2, c2, v3, c3)
    outs = pl.pallas_call(
        _edge_kernel_skip if skip_out else _edge_kernel_noskip,
        grid=(N_E // _R,),
        in_specs=[pl.BlockSpec((_R, din), lambda i: (i, 0))]
        + [pl.BlockSpec((_R, D), lambda i: (i, 0))] * 3
        + [_full(w) for w in ws],
        out_specs=[pl.BlockSpec((_R, D), lambda i: (i, 0))] * n_out,
        out_shape=[jax.ShapeDtypeStruct((N_E, D), jnp.float32)] * n_out,
    )(ea, xvs, xvd, xub, *ws)
    return outs if skip_out else (outs[0], outs[0])


def _phi_v_kernel_skip(xv, evp, degp, xun, xin, wa, wb, wc, bp, v2, c2, v3, c3,
                       o_pre, o_post):
    r = 1.0 / jnp.maximum(degp[0, :, :] + degp[1, :, :], 1.0)
    etov = (evp[0] + evp[1]) * r
    h = _sp(xv[...] @ wa[...] + etov @ wb[...] + xun[...] @ wc[...] + bp[...])
    h = _sp(h @ v2[...] + c2[...])
    v_pre = _sp(h @ v3[...] + c3[...])
    o_pre[...] = v_pre
    o_post[...] = v_pre + xin[...]


def _phi_v_kernel_noskip(xv, evp, degp, xun, wa, wb, wc, bp, v2, c2, v3, c3,
                         o_pre):
    r = 1.0 / jnp.maximum(degp[0, :, :] + degp[1, :, :], 1.0)
    etov = (evp[0] + evp[1]) * r
    h = _sp(xv[...] @ wa[...] + etov @ wb[...] + xun[...] @ wc[...] + bp[...])
    h = _sp(h @ v2[...] + c2[...])
    o_pre[...] = _sp(h @ v3[...] + c3[...])


def _phi_v(xv, evp, degp, xun, xin, p, skip_out):
    W = p["phi_v"][0]["W"]
    wa, wb, wc = W[0:D], W[D:2 * D], W[2 * D:3 * D]
    bp = p["phi_v"][0]["b"][None]
    v2, c2 = p["phi_v"][1]["W"], p["phi_v"][1]["b"][None]
    v3, c3 = p["phi_v"][2]["W"], p["phi_v"][2]["b"][None]
    degp3 = degp[:, :, None]
    ws = (wa, wb, wc, bp, v2, c2, v3, c3)
    n_out = 2 if skip_out else 1
    row = pl.BlockSpec((_R, D), lambda i: (i, 0))
    ins = [xv, evp, degp3, xun] + ([xin] if skip_out else [])
    in_specs = [row,
                pl.BlockSpec((2, _R, D), lambda i: (0, i, 0)),
                pl.BlockSpec((2, _R, 1), lambda i: (0, i, 0)),
                row] + ([row] if skip_out else [])
    outs = pl.pallas_call(
        _phi_v_kernel_skip if skip_out else _phi_v_kernel_noskip,
        grid=(N_V // _R,),
        in_specs=in_specs + [_full(w) for w in ws],
        out_specs=[row] * n_out,
        out_shape=[jax.ShapeDtypeStruct((N_V, D), jnp.float32)] * n_out,
    )(*ins, *ws)
    return outs if skip_out else (outs[0], outs[0])


def _phi_u_kernel(uep, uvp, cep, cvp, xu, uin, wa, wb, wc, bp, v2, c2, v3, c3,
                  o_ref):
    ue = (uep[0] + uep[1]) / jnp.maximum(cep[0] + cep[1], 1.0)
    uv = (uvp[0] + uvp[1]) / jnp.maximum(cvp[0] + cvp[1], 1.0)
    h = _sp(ue @ wa[...] + uv @ wb[...] + xu[...] @ wc[...] + bp[...])
    h = _sp(h @ v2[...] + c2[...])
    u_pre = _sp(h @ v3[...] + c3[...])
    o_ref[...] = u_pre + uin[...]


def _phi_u(uep, uvp, cep, cvp, xu, uin, p):
    W = p["phi_u"][0]["W"]
    wa, wb, wc = W[0:D], W[D:2 * D], W[2 * D:3 * D]
    bp = p["phi_u"][0]["b"][None]
    v2, c2 = p["phi_u"][1]["W"], p["phi_u"][1]["b"][None]
    v3, c3 = p["phi_u"][2]["W"], p["phi_u"][2]["b"][None]
    return pl.pallas_call(
        _phi_u_kernel,
        out_shape=jax.ShapeDtypeStruct((N_G, D), jnp.float32),
    )(uep, uvp, cep[:, :, None], cvp[:, :, None], xu, uin,
      wa, wb, wc, bp, v2, c2, v3, c3)


def _head_kernel(uv1, uv2, uv3, ue1, ue2, ue3, cep, cvp, uu3,
                 w0, b0, w1, b1, w2, b2, o_ref):
    mv_sum = uv1[0] + uv1[1] + uv2[0] + uv2[1] + uv3[0] + uv3[1]
    me_sum = ue1[0] + ue1[1] + ue2[0] + ue2[1] + ue3[0] + ue3[1]
    mv = mv_sum / jnp.maximum(cvp[0] + cvp[1], 1.0)
    me = me_sum / jnp.maximum(cep[0] + cep[1], 1.0)
    z = jnp.zeros_like(mv)
    tmp = jnp.concatenate([z, mv, z, me, uu3[...]], axis=1)
    h = _sp(tmp @ w0[...] + b0[...])
    h = _sp(h @ w1[...] + b1[...])
    o_ref[...] = h @ w2[...] + b2[...]


def _module(x, edge_attr, state_feat, uin, idx2d, counts, p, skip):
    src2d, dst2d, bb2d, nb2d, zeros = idx2d
    degp, cep, cvp = counts
    xv = _pre_v(x, p["pre_v"])
    xu = _pre_u(state_feat, p["pre_u"])
    xvs, xvd, xub, xun = _sc_gather(xv, xu, src2d, dst2d, bb2d, nb2d)
    e_pre, e_post = _edge_mlp(edge_attr, xvs, xvd, xub, p, skip)
    evp, uep = _sc_scatter_edges(e_pre, dst2d, bb2d, zeros)
    v_pre, x_post = _phi_v(xv, evp, degp, xun, x, p, skip)
    uvp = _sc_scatter_nodes(v_pre, nb2d, zeros)
    uu = _phi_u(uep, uvp, cep, cvp, xu, uin, p)
    return x_post, e_post, uu, uvp, uep


def kernel(x, edge_index, edge_attr, state, batch, bond_batch, params):
    src2d = edge_index[0].reshape(_E_CHUNKS * _E_K, _E_W)
    dst2d = edge_index[1].reshape(_E_CHUNKS * _E_K, _E_W)
    bb2d = bond_batch.reshape(_E_CHUNKS * _E_K, _E_W)
    nb2d = batch.reshape(_N_CHUNKS * _N_K, _N_W)
    zeros = jnp.zeros((N_V, D), jnp.float32)
    zeros1 = jnp.zeros((N_V,), jnp.float32)
    zg = jnp.zeros((N_G, D), jnp.float32)

    degp, cep, cvp = _sc_counts(dst2d, bb2d, nb2d, zeros1)
    counts = (degp, cep, cvp)
    idx2d = (src2d, dst2d, bb2d, nb2d, zeros)

    x1, ee1, uu1, uvp1, uep1 = _module(x, edge_attr, state, zg, idx2d,
                                       counts, params["m1"], False)
    x2, ee2, uu2, uvp2, uep2 = _module(x1, ee1, uu1, uu1, idx2d,
                                       counts, params["m2"], True)
    _, _, uu3, uvp3, uep3 = _module(x2, ee2, uu2, uu2, idx2d,
                                    counts, params["m3"], False)

    hl = params["hiddens"]
    out = pl.pallas_call(
        _head_kernel,
        out_shape=jax.ShapeDtypeStruct((N_G, 1), jnp.float32),
    )(uvp1, uvp2, uvp3, uep1, uep2, uep3, cep[:, :, None], cvp[:, :, None],
      uu3, hl[0]["W"], hl[0]["b"][None], hl[1]["W"], hl[1]["b"][None],
      hl[2]["W"], hl[2]["b"][None])
    return out
